# Initial kernel scaffold; baseline (speedup 1.0000x reference)
import functools

import jax
import jax.numpy as jnp
from jax import lax
from jax.experimental import pallas as pl
from jax.experimental.pallas import tpu as pltpu
from jax.experimental.pallas import tpu_sc as plsc

NN = 100000
EE = 3200000
NC, NS, LANES = 2, 16, 16
NW = NC * NS
NP = 102400
EPT = EE // NW
CHUNK = 800
NCH = EPT // CHUNK
NPT = NP // NS
TB = 1600
TG = NP // TB


def _sc_deg_body(col_hbm, ew_hbm, out_hbm, acc, idxb, valb, zb):
    cid = lax.axis_index("c")
    sid = lax.axis_index("s")
    wid = sid * NC + cid

    def zb_body(j, _):
        zb[pl.ds(j * LANES, LANES)] = jnp.zeros((LANES,), jnp.float32)
        return 0
    lax.fori_loop(0, CHUNK // LANES, zb_body, 0)

    def zs_body(j, _):
        pltpu.sync_copy(zb, acc.at[pl.ds(sid * NPT + j * CHUNK, CHUNK)])
        return 0
    lax.fori_loop(0, NPT // CHUNK, zs_body, 0)
    plsc.subcore_barrier()

    def ch_body(i, _):
        base = wid * EPT + i * CHUNK
        pltpu.sync_copy(col_hbm.at[pl.ds(base, CHUNK)], idxb)
        pltpu.sync_copy(ew_hbm.at[pl.ds(base, CHUNK)], valb)
        pltpu.sync_copy(valb, acc.at[idxb], add=True)
        return 0
    lax.fori_loop(0, NCH, ch_body, 0)
    plsc.subcore_barrier()

    def wb_body(j, _):
        off = sid * NPT + j * CHUNK
        pltpu.sync_copy(acc.at[pl.ds(off, CHUNK)], zb)
        pltpu.sync_copy(zb, out_hbm.at[pl.ds(cid * NP + off, CHUNK)])
        return 0
    lax.fori_loop(0, NPT // CHUNK, wb_body, 0)


_deg_call = pl.kernel(
    _sc_deg_body,
    out_type=jax.ShapeDtypeStruct((2 * NP,), jnp.float32),
    mesh=plsc.VectorSubcoreMesh(core_axis_name="c", subcore_axis_name="s"),
    scratch_types=[
        pltpu.VMEM_SHARED((NP,), jnp.float32),
        pltpu.VMEM((CHUNK,), jnp.int32),
        pltpu.VMEM((CHUNK,), jnp.float32),
        pltpu.VMEM((CHUNK,), jnp.float32),
    ],
)


def _sc_pass16_body(row_hbm, col_hbm, ew_hbm, zs_hbm, out_hbm,
                    acc, rowb, colb, ewb, rows, sem):
    cid = lax.axis_index("c")
    sid = lax.axis_index("s")
    wid = sid * NC + cid

    def zr(j, _):
        rows[j] = jnp.zeros((LANES,), jnp.float32)
        return 0
    lax.fori_loop(0, CHUNK, zr, 0)

    def zs_body(j, _):
        pltpu.sync_copy(rows, acc.at[pl.ds(sid * NPT + j * CHUNK, CHUNK)])
        return 0
    lax.fori_loop(0, NPT // CHUNK, zs_body, 0)
    plsc.subcore_barrier()

    def ch(i, _):
        base = wid * EPT + i * CHUNK
        pltpu.sync_copy(row_hbm.at[pl.ds(base, CHUNK)], rowb)
        pltpu.sync_copy(col_hbm.at[pl.ds(base, CHUNK)], colb)
        pltpu.sync_copy(ew_hbm.at[pl.ds(base, CHUNK)], ewb)
        pltpu.async_copy(zs_hbm.at[rowb], rows, sem).wait()

        def sc_body(j, _):
            b = j * 8
            for k in range(8):
                w = ewb[b + k]
                rows[b + k] = rows[b + k] * w
            return 0
        lax.fori_loop(0, CHUNK // 8, sc_body, 0)

        pltpu.sync_copy(rows, acc.at[colb], add=True)
        return 0
    lax.fori_loop(0, NCH, ch, 0)
    plsc.subcore_barrier()

    def wb(j, _):
        off = sid * NPT + j * CHUNK
        pltpu.sync_copy(acc.at[pl.ds(off, CHUNK)], rows)
        pltpu.sync_copy(rows, out_hbm.at[pl.ds(cid * NP + off, CHUNK)])
        return 0
    lax.fori_loop(0, NPT // CHUNK, wb, 0)


_pass16_call = pl.kernel(
    _sc_pass16_body,
    out_type=jax.ShapeDtypeStruct((2 * NP, 16), jnp.float32),
    mesh=plsc.VectorSubcoreMesh(core_axis_name="c", subcore_axis_name="s"),
    scratch_types=[
        pltpu.VMEM_SHARED((NP, 16), jnp.float32),
        pltpu.VMEM((CHUNK,), jnp.int32),
        pltpu.VMEM((CHUNK,), jnp.int32),
        pltpu.VMEM((CHUNK,), jnp.float32),
        pltpu.VMEM((CHUNK, 16), jnp.float32),
        pltpu.SemaphoreType.DMA,
    ],
)


def _sc_pass1_body(row_hbm, col_hbm, ew_hbm, zs3_hbm, out_hbm,
                   acc, table, rowb, colb, ewb, scl):
    cid = lax.axis_index("c")
    sid = lax.axis_index("s")
    wid = sid * NC + cid

    pltpu.sync_copy(zs3_hbm, table)

    def zb_body(j, _):
        scl[pl.ds(j * LANES, LANES)] = jnp.zeros((LANES,), jnp.float32)
        return 0
    lax.fori_loop(0, CHUNK // LANES, zb_body, 0)

    def zs_body(j, _):
        pltpu.sync_copy(scl, acc.at[pl.ds(sid * NPT + j * CHUNK, CHUNK)])
        return 0
    lax.fori_loop(0, NPT // CHUNK, zs_body, 0)
    plsc.subcore_barrier()

    def ch(i, _):
        base = wid * EPT + i * CHUNK
        pltpu.sync_copy(row_hbm.at[pl.ds(base, CHUNK)], rowb)
        pltpu.sync_copy(col_hbm.at[pl.ds(base, CHUNK)], colb)
        pltpu.sync_copy(ew_hbm.at[pl.ds(base, CHUNK)], ewb)

        def g(j, _):
            r = rowb[pl.ds(j * LANES, LANES)]
            v = plsc.load_gather(table, [r])
            w = ewb[pl.ds(j * LANES, LANES)]
            scl[pl.ds(j * LANES, LANES)] = v * w
            return 0
        lax.fori_loop(0, CHUNK // LANES, g, 0)

        pltpu.sync_copy(scl, acc.at[colb], add=True)
        return 0
    lax.fori_loop(0, NCH, ch, 0)
    plsc.subcore_barrier()

    def wb(j, _):
        off = sid * NPT + j * CHUNK
        pltpu.sync_copy(acc.at[pl.ds(off, CHUNK)], scl)
        pltpu.sync_copy(scl, out_hbm.at[pl.ds(cid * NP + off, CHUNK)])
        return 0
    lax.fori_loop(0, NPT // CHUNK, wb, 0)


_pass1_call = pl.kernel(
    _sc_pass1_body,
    out_type=jax.ShapeDtypeStruct((2 * NP,), jnp.float32),
    mesh=plsc.VectorSubcoreMesh(core_axis_name="c", subcore_axis_name="s"),
    scratch_types=[
        pltpu.VMEM_SHARED((NP,), jnp.float32),
        pltpu.VMEM((NP,), jnp.float32),
        pltpu.VMEM((CHUNK,), jnp.int32),
        pltpu.VMEM((CHUNK,), jnp.int32),
        pltpu.VMEM((CHUNK,), jnp.float32),
        pltpu.VMEM((CHUNK,), jnp.float32),
    ],
)


def _mm(h, W, K):
    out = h[:, 0:1] * W[0:1, :]
    for k in range(1, K):
        out = out + h[:, k:k + 1] * W[k:k + 1, :]
    return out


def _t1_body(deg0_ref, deg1_ref, x_ref, w1_ref, dis_ref, zs1_ref):
    deg = deg0_ref[...] + deg1_ref[...] + 2.0
    dis = jnp.where(deg > 0, lax.rsqrt(deg), 0.0)
    dis_ref[...] = dis
    zs1_ref[...] = dis * _mm(x_ref[...], w1_ref[...], 3)


_t1_call = pl.pallas_call(
    _t1_body,
    grid=(TG,),
    in_specs=[
        pl.BlockSpec((TB, 1), lambda i: (i, 0)),
        pl.BlockSpec((TB, 1), lambda i: (i, 0)),
        pl.BlockSpec((TB, 3), lambda i: (i, 0)),
        pl.BlockSpec((3, 16), lambda i: (0, 0)),
    ],
    out_specs=[
        pl.BlockSpec((TB, 1), lambda i: (i, 0)),
        pl.BlockSpec((TB, 16), lambda i: (i, 0)),
    ],
    out_shape=[
        jax.ShapeDtypeStruct((NP, 1), jnp.float32),
        jax.ShapeDtypeStruct((NP, 16), jnp.float32),
    ],
)


def _t2_body(a0_ref, a1_ref, zs_ref, dis_ref, b_ref, w_ref, zsn_ref):
    pre = dis_ref[...] * (a0_ref[...] + a1_ref[...] + 2.0 * zs_ref[...]) + b_ref[...]
    h = jnp.maximum(pre, 0.0)
    zsn_ref[...] = dis_ref[...] * _mm(h, w_ref[...], 16)


_t2_call = pl.pallas_call(
    _t2_body,
    grid=(TG,),
    in_specs=[
        pl.BlockSpec((TB, 16), lambda i: (i, 0)),
        pl.BlockSpec((TB, 16), lambda i: (i, 0)),
        pl.BlockSpec((TB, 16), lambda i: (i, 0)),
        pl.BlockSpec((TB, 1), lambda i: (i, 0)),
        pl.BlockSpec((1, 16), lambda i: (0, 0)),
        pl.BlockSpec((16, 16), lambda i: (0, 0)),
    ],
    out_specs=pl.BlockSpec((TB, 16), lambda i: (i, 0)),
    out_shape=jax.ShapeDtypeStruct((NP, 16), jnp.float32),
)


def _t3_body(a0_ref, a1_ref, zs_ref, dis_ref, b_ref, w3_ref, zs3_ref, ps_ref):
    i = pl.program_id(0)
    pre = dis_ref[...] * (a0_ref[...] + a1_ref[...] + 2.0 * zs_ref[...]) + b_ref[...]
    h = jnp.maximum(pre, 0.0)
    ridx = lax.broadcasted_iota(jnp.int32, (TB, 1), 0) + i * TB
    hm = jnp.where(ridx < NN, h, 0.0)
    zs3_ref[...] = dis_ref[...] * _mm(h, w3_ref[...], 16)

    @pl.when(i == 0)
    def _():
        ps_ref[...] = jnp.zeros((1, 16), jnp.float32)

    ps_ref[...] += jnp.sum(hm, axis=0, keepdims=True)


_t3_call = pl.pallas_call(
    _t3_body,
    grid=(TG,),
    in_specs=[
        pl.BlockSpec((TB, 16), lambda i: (i, 0)),
        pl.BlockSpec((TB, 16), lambda i: (i, 0)),
        pl.BlockSpec((TB, 16), lambda i: (i, 0)),
        pl.BlockSpec((TB, 1), lambda i: (i, 0)),
        pl.BlockSpec((1, 16), lambda i: (0, 0)),
        pl.BlockSpec((16, 1), lambda i: (0, 0)),
    ],
    out_specs=[
        pl.BlockSpec((TB, 1), lambda i: (i, 0)),
        pl.BlockSpec((1, 16), lambda i: (0, 0)),
    ],
    out_shape=[
        jax.ShapeDtypeStruct((NP, 1), jnp.float32),
        jax.ShapeDtypeStruct((1, 16), jnp.float32),
    ],
)


def _kc_body(a0_ref, a1_ref, zs3_ref, dis_ref, b3_ref, ch_ref, lg_ref, m_ref):
    i = pl.program_id(0)
    c = dis_ref[...] * (a0_ref[...] + a1_ref[...] + 2.0 * zs3_ref[...]) + b3_ref[...]
    lg = jnp.where(ch_ref[...], c, -1e9)
    lg_ref[...] = lg

    @pl.when(i == 0)
    def _():
        m_ref[...] = jnp.full((1, 1), -3e38, jnp.float32)

    m_ref[...] = jnp.maximum(m_ref[...], jnp.max(lg, keepdims=True))


_kc_call = pl.pallas_call(
    _kc_body,
    grid=(TG,),
    in_specs=[
        pl.BlockSpec((TB, 1), lambda i: (i, 0)),
        pl.BlockSpec((TB, 1), lambda i: (i, 0)),
        pl.BlockSpec((TB, 1), lambda i: (i, 0)),
        pl.BlockSpec((TB, 1), lambda i: (i, 0)),
        pl.BlockSpec((1, 1), lambda i: (0, 0)),
        pl.BlockSpec((TB, 1), lambda i: (i, 0)),
    ],
    out_specs=[
        pl.BlockSpec((TB, 1), lambda i: (i, 0)),
        pl.BlockSpec((1, 1), lambda i: (0, 0)),
    ],
    out_shape=[
        jax.ShapeDtypeStruct((NP, 1), jnp.float32),
        jax.ShapeDtypeStruct((1, 1), jnp.float32),
    ],
)


def _ke_body(lg_ref, m_ref, e_ref, s_ref):
    i = pl.program_id(0)
    e = jnp.exp(lg_ref[...] - m_ref[...])
    e_ref[...] = e

    @pl.when(i == 0)
    def _():
        s_ref[...] = jnp.zeros((1, 1), jnp.float32)

    s_ref[...] += jnp.sum(e, keepdims=True)


_ke_call = pl.pallas_call(
    _ke_body,
    grid=(TG,),
    in_specs=[
        pl.BlockSpec((TB, 1), lambda i: (i, 0)),
        pl.BlockSpec((1, 1), lambda i: (0, 0)),
    ],
    out_specs=[
        pl.BlockSpec((TB, 1), lambda i: (i, 0)),
        pl.BlockSpec((1, 1), lambda i: (0, 0)),
    ],
    out_shape=[
        jax.ShapeDtypeStruct((NP, 1), jnp.float32),
        jax.ShapeDtypeStruct((1, 1), jnp.float32),
    ],
)


def _kf_body(e_ref, s_ref, ch_ref, ps_ref, fcw_ref, fcb_ref, choice_ref, val_ref):
    i = pl.program_id(0)
    p = e_ref[...] / s_ref[...]
    choice_ref[...] = jnp.where(ch_ref[...], p, 0.0)

    @pl.when(i == 0)
    def _():
        v = ps_ref[...] * (1.0 / NN)
        val_ref[...] = _mm(v, fcw_ref[...], 16) + fcb_ref[...]


_kf_call = pl.pallas_call(
    _kf_body,
    grid=(TG,),
    in_specs=[
        pl.BlockSpec((TB, 1), lambda i: (i, 0)),
        pl.BlockSpec((1, 1), lambda i: (0, 0)),
        pl.BlockSpec((TB, 1), lambda i: (i, 0)),
        pl.BlockSpec((1, 16), lambda i: (0, 0)),
        pl.BlockSpec((16, 1), lambda i: (0, 0)),
        pl.BlockSpec((1, 1), lambda i: (0, 0)),
    ],
    out_specs=[
        pl.BlockSpec((TB, 1), lambda i: (i, 0)),
        pl.BlockSpec((1, 1), lambda i: (0, 0)),
    ],
    out_shape=[
        jax.ShapeDtypeStruct((NP, 1), jnp.float32),
        jax.ShapeDtypeStruct((1, 1), jnp.float32),
    ],
)


@jax.jit
def _run(x, edge_attr, W1, b1, W2, b2, W3, b3, fc_W, fc_b, edge_index, choices):
    row = edge_index[0]
    col = edge_index[1]
    xp = jnp.pad(x, ((0, NP - NN), (0, 0)))
    chp = jnp.pad(choices, (0, NP - NN)).reshape(NP, 1)

    degp = _deg_call(col, edge_attr)
    deg0 = degp[:NP].reshape(NP, 1)
    deg1 = degp[NP:].reshape(NP, 1)

    dis, zs1 = _t1_call(deg0, deg1, xp, W1)
    acc1 = _pass16_call(row, col, edge_attr, zs1)
    zs2 = _t2_call(acc1[:NP], acc1[NP:], zs1, dis, b1.reshape(1, 16), W2)
    acc2 = _pass16_call(row, col, edge_attr, zs2)
    zs3, ps = _t3_call(acc2[:NP], acc2[NP:], zs2, dis, b2.reshape(1, 16), W3)
    cacc = _pass1_call(row, col, edge_attr, zs3.reshape(NP))

    lg, m = _kc_call(cacc[:NP].reshape(NP, 1), cacc[NP:].reshape(NP, 1),
                     zs3, dis, b3.reshape(1, 1), chp)
    e, s = _ke_call(lg, m)
    choice, value = _kf_call(e, s, chp, ps, fc_W, fc_b.reshape(1, 1))
    return choice[:NN, 0], value


def kernel(x, edge_attr, W1, b1, W2, b2, W3, b3, fc_W, fc_b, edge_index, choices):
    return _run(x, edge_attr, W1, b1, W2, b2, W3, b3, fc_W, fc_b,
                edge_index, choices)


# speedup vs baseline: 35.9139x; 35.9139x over previous
import functools

import jax
import jax.numpy as jnp
from jax import lax
from jax.experimental import pallas as pl
from jax.experimental.pallas import tpu as pltpu
from jax.experimental.pallas import tpu_sc as plsc

NN = 100000
EE = 3200000
NC, NS, LANES = 2, 16, 16
NW = NC * NS
NP = 102400
EPT = EE // NW
CHUNK = 800
NCH = EPT // CHUNK
NPT = NP // NS
TB = 1600
TG = NP // TB


def _sc_deg_body(col_hbm, ew_hbm, out_hbm, acc, idxb, valb, zb):
    cid = lax.axis_index("c")
    sid = lax.axis_index("s")
    wid = sid * NC + cid

    def zb_body(j, _):
        zb[pl.ds(j * LANES, LANES)] = jnp.zeros((LANES,), jnp.float32)
        return 0
    lax.fori_loop(0, CHUNK // LANES, zb_body, 0)

    def zs_body(j, _):
        pltpu.sync_copy(zb, acc.at[pl.ds(sid * NPT + j * CHUNK, CHUNK)])
        return 0
    lax.fori_loop(0, NPT // CHUNK, zs_body, 0)
    plsc.subcore_barrier()

    def ch_body(i, _):
        base = wid * EPT + i * CHUNK
        pltpu.sync_copy(col_hbm.at[pl.ds(base, CHUNK)], idxb)
        pltpu.sync_copy(ew_hbm.at[pl.ds(base, CHUNK)], valb)
        pltpu.sync_copy(valb, acc.at[idxb], add=True)
        return 0
    lax.fori_loop(0, NCH, ch_body, 0)
    plsc.subcore_barrier()

    def wb_body(j, _):
        off = sid * NPT + j * CHUNK
        pltpu.sync_copy(acc.at[pl.ds(off, CHUNK)], zb)
        pltpu.sync_copy(zb, out_hbm.at[pl.ds(cid * NP + off, CHUNK)])
        return 0
    lax.fori_loop(0, NPT // CHUNK, wb_body, 0)


_deg_call = pl.kernel(
    _sc_deg_body,
    out_type=jax.ShapeDtypeStruct((2 * NP,), jnp.float32),
    mesh=plsc.VectorSubcoreMesh(core_axis_name="c", subcore_axis_name="s"),
    compiler_params=pltpu.CompilerParams(use_tc_tiling_on_sc=False, needs_layout_passes=False),
    scratch_types=[
        pltpu.VMEM_SHARED((NP,), jnp.float32),
        pltpu.VMEM((CHUNK,), jnp.int32),
        pltpu.VMEM((CHUNK,), jnp.float32),
        pltpu.VMEM((CHUNK,), jnp.float32),
    ],
)


def _sc_pass16_body(row_hbm, col_hbm, ew_hbm, zs_hbm, out_hbm,
                    acc, rowb, colb, ewb, rows, sem):
    cid = lax.axis_index("c")
    sid = lax.axis_index("s")
    wid = sid * NC + cid

    def zr(j, _):
        rows[j] = jnp.zeros((LANES,), jnp.float32)
        return 0
    lax.fori_loop(0, CHUNK, zr, 0)

    def zs_body(j, _):
        pltpu.sync_copy(rows, acc.at[pl.ds(sid * NPT + j * CHUNK, CHUNK)])
        return 0
    lax.fori_loop(0, NPT // CHUNK, zs_body, 0)
    plsc.subcore_barrier()

    def ch(i, _):
        base = wid * EPT + i * CHUNK
        pltpu.sync_copy(row_hbm.at[pl.ds(base, CHUNK)], rowb)
        pltpu.sync_copy(col_hbm.at[pl.ds(base, CHUNK)], colb)
        pltpu.sync_copy(ew_hbm.at[pl.ds(base, CHUNK)], ewb)
        pltpu.async_copy(zs_hbm.at[rowb], rows, sem).wait()

        def sc_body(j, _):
            b = j * LANES
            wv = ewb[pl.ds(b, LANES)]
            for k in range(LANES):
                rows[b + k] = rows[b + k] * wv[k]
            return 0
        lax.fori_loop(0, CHUNK // LANES, sc_body, 0)

        pltpu.sync_copy(rows, acc.at[colb], add=True)
        return 0
    lax.fori_loop(0, NCH, ch, 0)
    plsc.subcore_barrier()

    def wb(j, _):
        off = sid * NPT + j * CHUNK
        pltpu.sync_copy(acc.at[pl.ds(off, CHUNK)], rows)
        pltpu.sync_copy(rows, out_hbm.at[pl.ds(cid * NP + off, CHUNK)])
        return 0
    lax.fori_loop(0, NPT // CHUNK, wb, 0)


_pass16_call = pl.kernel(
    _sc_pass16_body,
    out_type=jax.ShapeDtypeStruct((2 * NP, 16), jnp.float32),
    mesh=plsc.VectorSubcoreMesh(core_axis_name="c", subcore_axis_name="s"),
    compiler_params=pltpu.CompilerParams(use_tc_tiling_on_sc=False, needs_layout_passes=False),
    scratch_types=[
        pltpu.VMEM_SHARED((NP, 16), jnp.float32),
        pltpu.VMEM((CHUNK,), jnp.int32),
        pltpu.VMEM((CHUNK,), jnp.int32),
        pltpu.VMEM((CHUNK,), jnp.float32),
        pltpu.VMEM((CHUNK, 16), jnp.float32),
        pltpu.SemaphoreType.DMA,
    ],
)


def _sc_pass1_body(row_hbm, col_hbm, ew_hbm, zs3_hbm, out_hbm,
                   acc, table, rowb, colb, ewb, scl):
    cid = lax.axis_index("c")
    sid = lax.axis_index("s")
    wid = sid * NC + cid

    pltpu.sync_copy(zs3_hbm, table)

    def zb_body(j, _):
        scl[pl.ds(j * LANES, LANES)] = jnp.zeros((LANES,), jnp.float32)
        return 0
    lax.fori_loop(0, CHUNK // LANES, zb_body, 0)

    def zs_body(j, _):
        pltpu.sync_copy(scl, acc.at[pl.ds(sid * NPT + j * CHUNK, CHUNK)])
        return 0
    lax.fori_loop(0, NPT // CHUNK, zs_body, 0)
    plsc.subcore_barrier()

    def ch(i, _):
        base = wid * EPT + i * CHUNK
        pltpu.sync_copy(row_hbm.at[pl.ds(base, CHUNK)], rowb)
        pltpu.sync_copy(col_hbm.at[pl.ds(base, CHUNK)], colb)
        pltpu.sync_copy(ew_hbm.at[pl.ds(base, CHUNK)], ewb)

        def g(j, _):
            r = rowb[pl.ds(j * LANES, LANES)]
            v = plsc.load_gather(table, [r])
            w = ewb[pl.ds(j * LANES, LANES)]
            scl[pl.ds(j * LANES, LANES)] = v * w
            return 0
        lax.fori_loop(0, CHUNK // LANES, g, 0)

        pltpu.sync_copy(scl, acc.at[colb], add=True)
        return 0
    lax.fori_loop(0, NCH, ch, 0)
    plsc.subcore_barrier()

    def wb(j, _):
        off = sid * NPT + j * CHUNK
        pltpu.sync_copy(acc.at[pl.ds(off, CHUNK)], scl)
        pltpu.sync_copy(scl, out_hbm.at[pl.ds(cid * NP + off, CHUNK)])
        return 0
    lax.fori_loop(0, NPT // CHUNK, wb, 0)


_pass1_call = pl.kernel(
    _sc_pass1_body,
    out_type=jax.ShapeDtypeStruct((2 * NP,), jnp.float32),
    mesh=plsc.VectorSubcoreMesh(core_axis_name="c", subcore_axis_name="s"),
    compiler_params=pltpu.CompilerParams(use_tc_tiling_on_sc=False, needs_layout_passes=False),
    scratch_types=[
        pltpu.VMEM_SHARED((NP,), jnp.float32),
        pltpu.VMEM((NP,), jnp.float32),
        pltpu.VMEM((CHUNK,), jnp.int32),
        pltpu.VMEM((CHUNK,), jnp.int32),
        pltpu.VMEM((CHUNK,), jnp.float32),
        pltpu.VMEM((CHUNK,), jnp.float32),
    ],
)


def _mm(h, W, K):
    out = h[:, 0:1] * W[0:1, :]
    for k in range(1, K):
        out = out + h[:, k:k + 1] * W[k:k + 1, :]
    return out


def _t1_body(deg0_ref, deg1_ref, x_ref, w1_ref, dis_ref, zs1_ref):
    deg = deg0_ref[...] + deg1_ref[...] + 2.0
    dis = jnp.where(deg > 0, lax.rsqrt(deg), 0.0)
    dis_ref[...] = dis
    zs1_ref[...] = dis * _mm(x_ref[...], w1_ref[...], 3)


_t1_call = pl.pallas_call(
    _t1_body,
    grid=(TG,),
    in_specs=[
        pl.BlockSpec((TB, 1), lambda i: (i, 0)),
        pl.BlockSpec((TB, 1), lambda i: (i, 0)),
        pl.BlockSpec((TB, 3), lambda i: (i, 0)),
        pl.BlockSpec((3, 16), lambda i: (0, 0)),
    ],
    out_specs=[
        pl.BlockSpec((TB, 1), lambda i: (i, 0)),
        pl.BlockSpec((TB, 16), lambda i: (i, 0)),
    ],
    out_shape=[
        jax.ShapeDtypeStruct((NP, 1), jnp.float32),
        jax.ShapeDtypeStruct((NP, 16), jnp.float32),
    ],
)


def _t2_body(a0_ref, a1_ref, zs_ref, dis_ref, b_ref, w_ref, zsn_ref):
    pre = dis_ref[...] * (a0_ref[...] + a1_ref[...] + 2.0 * zs_ref[...]) + b_ref[...]
    h = jnp.maximum(pre, 0.0)
    zsn_ref[...] = dis_ref[...] * _mm(h, w_ref[...], 16)


_t2_call = pl.pallas_call(
    _t2_body,
    grid=(TG,),
    in_specs=[
        pl.BlockSpec((TB, 16), lambda i: (i, 0)),
        pl.BlockSpec((TB, 16), lambda i: (i, 0)),
        pl.BlockSpec((TB, 16), lambda i: (i, 0)),
        pl.BlockSpec((TB, 1), lambda i: (i, 0)),
        pl.BlockSpec((1, 16), lambda i: (0, 0)),
        pl.BlockSpec((16, 16), lambda i: (0, 0)),
    ],
    out_specs=pl.BlockSpec((TB, 16), lambda i: (i, 0)),
    out_shape=jax.ShapeDtypeStruct((NP, 16), jnp.float32),
)


def _t3_body(a0_ref, a1_ref, zs_ref, dis_ref, b_ref, w3_ref, zs3_ref, ps_ref):
    i = pl.program_id(0)
    pre = dis_ref[...] * (a0_ref[...] + a1_ref[...] + 2.0 * zs_ref[...]) + b_ref[...]
    h = jnp.maximum(pre, 0.0)
    ridx = lax.broadcasted_iota(jnp.int32, (TB, 1), 0) + i * TB
    hm = jnp.where(ridx < NN, h, 0.0)
    zs3_ref[...] = dis_ref[...] * _mm(h, w3_ref[...], 16)

    @pl.when(i == 0)
    def _():
        ps_ref[...] = jnp.zeros((1, 16), jnp.float32)

    ps_ref[...] += jnp.sum(hm, axis=0, keepdims=True)


_t3_call = pl.pallas_call(
    _t3_body,
    grid=(TG,),
    in_specs=[
        pl.BlockSpec((TB, 16), lambda i: (i, 0)),
        pl.BlockSpec((TB, 16), lambda i: (i, 0)),
        pl.BlockSpec((TB, 16), lambda i: (i, 0)),
        pl.BlockSpec((TB, 1), lambda i: (i, 0)),
        pl.BlockSpec((1, 16), lambda i: (0, 0)),
        pl.BlockSpec((16, 1), lambda i: (0, 0)),
    ],
    out_specs=[
        pl.BlockSpec((TB, 1), lambda i: (i, 0)),
        pl.BlockSpec((1, 16), lambda i: (0, 0)),
    ],
    out_shape=[
        jax.ShapeDtypeStruct((NP, 1), jnp.float32),
        jax.ShapeDtypeStruct((1, 16), jnp.float32),
    ],
)


def _kc_body(a0_ref, a1_ref, zs3_ref, dis_ref, b3_ref, ch_ref, lg_ref, m_ref):
    i = pl.program_id(0)
    c = dis_ref[...] * (a0_ref[...] + a1_ref[...] + 2.0 * zs3_ref[...]) + b3_ref[...]
    lg = jnp.where(ch_ref[...], c, -1e9)
    lg_ref[...] = lg

    @pl.when(i == 0)
    def _():
        m_ref[...] = jnp.full((1, 1), -3e38, jnp.float32)

    m_ref[...] = jnp.maximum(m_ref[...], jnp.max(lg, keepdims=True))


_kc_call = pl.pallas_call(
    _kc_body,
    grid=(TG,),
    in_specs=[
        pl.BlockSpec((TB, 1), lambda i: (i, 0)),
        pl.BlockSpec((TB, 1), lambda i: (i, 0)),
        pl.BlockSpec((TB, 1), lambda i: (i, 0)),
        pl.BlockSpec((TB, 1), lambda i: (i, 0)),
        pl.BlockSpec((1, 1), lambda i: (0, 0)),
        pl.BlockSpec((TB, 1), lambda i: (i, 0)),
    ],
    out_specs=[
        pl.BlockSpec((TB, 1), lambda i: (i, 0)),
        pl.BlockSpec((1, 1), lambda i: (0, 0)),
    ],
    out_shape=[
        jax.ShapeDtypeStruct((NP, 1), jnp.float32),
        jax.ShapeDtypeStruct((1, 1), jnp.float32),
    ],
)


def _ke_body(lg_ref, m_ref, e_ref, s_ref):
    i = pl.program_id(0)
    e = jnp.exp(lg_ref[...] - m_ref[...])
    e_ref[...] = e

    @pl.when(i == 0)
    def _():
        s_ref[...] = jnp.zeros((1, 1), jnp.float32)

    s_ref[...] += jnp.sum(e, keepdims=True)


_ke_call = pl.pallas_call(
    _ke_body,
    grid=(TG,),
    in_specs=[
        pl.BlockSpec((TB, 1), lambda i: (i, 0)),
        pl.BlockSpec((1, 1), lambda i: (0, 0)),
    ],
    out_specs=[
        pl.BlockSpec((TB, 1), lambda i: (i, 0)),
        pl.BlockSpec((1, 1), lambda i: (0, 0)),
    ],
    out_shape=[
        jax.ShapeDtypeStruct((NP, 1), jnp.float32),
        jax.ShapeDtypeStruct((1, 1), jnp.float32),
    ],
)


def _kf_body(e_ref, s_ref, ch_ref, ps_ref, fcw_ref, fcb_ref, choice_ref, val_ref):
    i = pl.program_id(0)
    p = e_ref[...] / s_ref[...]
    choice_ref[...] = jnp.where(ch_ref[...], p, 0.0)

    @pl.when(i == 0)
    def _():
        v = ps_ref[...] * (1.0 / NN)
        val_ref[...] = _mm(v, fcw_ref[...], 16) + fcb_ref[...]


_kf_call = pl.pallas_call(
    _kf_body,
    grid=(TG,),
    in_specs=[
        pl.BlockSpec((TB, 1), lambda i: (i, 0)),
        pl.BlockSpec((1, 1), lambda i: (0, 0)),
        pl.BlockSpec((TB, 1), lambda i: (i, 0)),
        pl.BlockSpec((1, 16), lambda i: (0, 0)),
        pl.BlockSpec((16, 1), lambda i: (0, 0)),
        pl.BlockSpec((1, 1), lambda i: (0, 0)),
    ],
    out_specs=[
        pl.BlockSpec((TB, 1), lambda i: (i, 0)),
        pl.BlockSpec((1, 1), lambda i: (0, 0)),
    ],
    out_shape=[
        jax.ShapeDtypeStruct((NP, 1), jnp.float32),
        jax.ShapeDtypeStruct((1, 1), jnp.float32),
    ],
)


@jax.jit
def _run(x, edge_attr, W1, b1, W2, b2, W3, b3, fc_W, fc_b, edge_index, choices):
    row = edge_index[0]
    col = edge_index[1]
    xp = jnp.pad(x, ((0, NP - NN), (0, 0)))
    chp = jnp.pad(choices, (0, NP - NN)).reshape(NP, 1)

    degp = _deg_call(col, edge_attr)
    deg0 = degp[:NP].reshape(NP, 1)
    deg1 = degp[NP:].reshape(NP, 1)

    dis, zs1 = _t1_call(deg0, deg1, xp, W1)
    acc1 = _pass16_call(row, col, edge_attr, zs1)
    zs2 = _t2_call(acc1[:NP], acc1[NP:], zs1, dis, b1.reshape(1, 16), W2)
    acc2 = _pass16_call(row, col, edge_attr, zs2)
    zs3, ps = _t3_call(acc2[:NP], acc2[NP:], zs2, dis, b2.reshape(1, 16), W3)
    cacc = _pass1_call(row, col, edge_attr, zs3.reshape(NP))

    lg, m = _kc_call(cacc[:NP].reshape(NP, 1), cacc[NP:].reshape(NP, 1),
                     zs3, dis, b3.reshape(1, 1), chp)
    e, s = _ke_call(lg, m)
    choice, value = _kf_call(e, s, chp, ps, fc_W, fc_b.reshape(1, 1))
    return choice[:NN, 0], value


def kernel(x, edge_attr, W1, b1, W2, b2, W3, b3, fc_W, fc_b, edge_index, choices):
    return _run(x, edge_attr, W1, b1, W2, b2, W3, b3, fc_W, fc_b,
                edge_index, choices)


# speedup vs baseline: 46.4517x; 1.2934x over previous
import jax
import jax.numpy as jnp
from jax import lax
from jax.experimental import pallas as pl
from jax.experimental.pallas import tpu as pltpu
from jax.experimental.pallas import tpu_sc as plsc

NN = 100000
EE = 3200000
NC, NS, LANES = 2, 16, 16
NW = NC * NS
NP = 102400
EPT = EE // NW
CHUNK = 800
NCH = EPT // CHUNK
NPT = NP // NS
WCH = 800
RW = NP // 8
BRW = 1600
RC = NP // 128
BRC = 200

_SC_PARAMS = pltpu.CompilerParams(use_tc_tiling_on_sc=False,
                                  needs_layout_passes=False)
_MESH = dict(mesh=plsc.VectorSubcoreMesh(core_axis_name="c",
                                         subcore_axis_name="s"),
             compiler_params=_SC_PARAMS)


def _sc_deg_body(col_hbm, ew_hbm, outw_hbm, outc_hbm, acc, idxb, valb, zb, wbuf):
    cid = lax.axis_index("c")
    sid = lax.axis_index("s")
    wid = sid * NC + cid

    def zb_body(j, _):
        zb[pl.ds(j * LANES, LANES)] = jnp.zeros((LANES,), jnp.float32)
        return 0
    lax.fori_loop(0, WCH // LANES, zb_body, 0)

    def zs_body(j, _):
        pltpu.sync_copy(zb, acc.at[pl.ds(sid * NPT + j * WCH, WCH)])
        return 0
    lax.fori_loop(0, NPT // WCH, zs_body, 0)
    plsc.subcore_barrier()

    def ch_body(i, _):
        base = wid * EPT + i * CHUNK
        pltpu.sync_copy(col_hbm.at[pl.ds(base, CHUNK)], idxb)
        pltpu.sync_copy(ew_hbm.at[pl.ds(base, CHUNK)], valb)
        pltpu.sync_copy(valb, acc.at[idxb], add=True)
        return 0
    lax.fori_loop(0, NCH, ch_body, 0)
    plsc.subcore_barrier()

    def wb_body(j, _):
        off = sid * NPT + j * WCH
        pltpu.sync_copy(acc.at[pl.ds(off, WCH)], zb)
        pltpu.sync_copy(zb, outc_hbm.at[pl.ds(cid * NP + off, WCH)])

        def expand(g, _):
            dv = zb[pl.ds(g * LANES, LANES)]
            for k in range(LANES):
                wbuf[g * LANES + k] = jnp.full((LANES,), dv[k], jnp.float32)
            return 0
        lax.fori_loop(0, WCH // LANES, expand, 0)
        pltpu.sync_copy(wbuf, outw_hbm.at[pl.ds(cid * NP + off, WCH)])
        return 0
    lax.fori_loop(0, NPT // WCH, wb_body, 0)


_deg_call = pl.kernel(
    _sc_deg_body,
    out_type=[jax.ShapeDtypeStruct((2 * NP, 16), jnp.float32),
              jax.ShapeDtypeStruct((2 * NP,), jnp.float32)],
    scratch_types=[
        pltpu.VMEM_SHARED((NP,), jnp.float32),
        pltpu.VMEM((CHUNK,), jnp.int32),
        pltpu.VMEM((CHUNK,), jnp.float32),
        pltpu.VMEM((WCH,), jnp.float32),
        pltpu.VMEM((WCH, 16), jnp.float32),
    ],
    **_MESH,
)


def _sc_pass16_body(row_hbm, col_hbm, ew_hbm, zs_hbm, out_hbm,
                    acc, rowb, colb, ewb, rows, sem):
    cid = lax.axis_index("c")
    sid = lax.axis_index("s")
    wid = sid * NC + cid

    def zr(j, _):
        rows[j] = jnp.zeros((LANES,), jnp.float32)
        return 0
    lax.fori_loop(0, WCH, zr, 0)

    def zs_body(j, _):
        pltpu.sync_copy(rows.at[pl.ds(0, WCH)],
                        acc.at[pl.ds(sid * NPT + j * WCH, WCH)])
        return 0
    lax.fori_loop(0, NPT // WCH, zs_body, 0)
    plsc.subcore_barrier()

    def ch(i, _):
        base = wid * EPT + i * CHUNK
        pltpu.sync_copy(row_hbm.at[pl.ds(base, CHUNK)], rowb)
        pltpu.sync_copy(col_hbm.at[pl.ds(base, CHUNK)], colb)
        pltpu.sync_copy(ew_hbm.at[pl.ds(base, CHUNK)], ewb)
        pltpu.async_copy(zs_hbm.at[rowb], rows, sem).wait()

        def sc_body(j, _):
            b = j * LANES
            wv = ewb[pl.ds(b, LANES)]
            for k in range(LANES):
                rows[b + k] = rows[b + k] * wv[k]
            return 0
        lax.fori_loop(0, CHUNK // LANES, sc_body, 0)

        pltpu.sync_copy(rows, acc.at[colb], add=True)
        return 0
    lax.fori_loop(0, NCH, ch, 0)
    plsc.subcore_barrier()

    def wb(j, _):
        off = sid * NPT + j * WCH
        pltpu.sync_copy(acc.at[pl.ds(off, WCH)], rows.at[pl.ds(0, WCH)])
        pltpu.sync_copy(rows.at[pl.ds(0, WCH)],
                        out_hbm.at[pl.ds(cid * NP + off, WCH)])
        return 0
    lax.fori_loop(0, NPT // WCH, wb, 0)


_pass16_call = pl.kernel(
    _sc_pass16_body,
    out_type=jax.ShapeDtypeStruct((2 * NP, 16), jnp.float32),
    scratch_types=[
        pltpu.VMEM_SHARED((NP, 16), jnp.float32),
        pltpu.VMEM((CHUNK,), jnp.int32),
        pltpu.VMEM((CHUNK,), jnp.int32),
        pltpu.VMEM((CHUNK,), jnp.float32),
        pltpu.VMEM((CHUNK, 16), jnp.float32),
        pltpu.SemaphoreType.DMA,
    ],
    **_MESH,
)


def _sc_pass1_body(row_hbm, col_hbm, ew_hbm, zs3_hbm, out_hbm,
                   acc, table, rowb, colb, ewb, scl):
    cid = lax.axis_index("c")
    sid = lax.axis_index("s")
    wid = sid * NC + cid

    pltpu.sync_copy(zs3_hbm, table)

    def zb_body(j, _):
        scl[pl.ds(j * LANES, LANES)] = jnp.zeros((LANES,), jnp.float32)
        return 0
    lax.fori_loop(0, WCH // LANES, zb_body, 0)

    def zs_body(j, _):
        pltpu.sync_copy(scl.at[pl.ds(0, WCH)],
                        acc.at[pl.ds(sid * NPT + j * WCH, WCH)])
        return 0
    lax.fori_loop(0, NPT // WCH, zs_body, 0)
    plsc.subcore_barrier()

    def ch(i, _):
        base = wid * EPT + i * CHUNK
        pltpu.sync_copy(row_hbm.at[pl.ds(base, CHUNK)], rowb)
        pltpu.sync_copy(col_hbm.at[pl.ds(base, CHUNK)], colb)
        pltpu.sync_copy(ew_hbm.at[pl.ds(base, CHUNK)], ewb)

        def g(j, _):
            r = rowb[pl.ds(j * LANES, LANES)]
            v = plsc.load_gather(table, [r])
            w = ewb[pl.ds(j * LANES, LANES)]
            scl[pl.ds(j * LANES, LANES)] = v * w
            return 0
        lax.fori_loop(0, CHUNK // LANES, g, 0)

        pltpu.sync_copy(scl, acc.at[colb], add=True)
        return 0
    lax.fori_loop(0, NCH, ch, 0)
    plsc.subcore_barrier()

    def wb(j, _):
        off = sid * NPT + j * WCH
        pltpu.sync_copy(acc.at[pl.ds(off, WCH)], scl.at[pl.ds(0, WCH)])
        pltpu.sync_copy(scl.at[pl.ds(0, WCH)],
                        out_hbm.at[pl.ds(cid * NP + off, WCH)])
        return 0
    lax.fori_loop(0, NPT // WCH, wb, 0)


_pass1_call = pl.kernel(
    _sc_pass1_body,
    out_type=jax.ShapeDtypeStruct((2 * NP,), jnp.float32),
    scratch_types=[
        pltpu.VMEM_SHARED((NP,), jnp.float32),
        pltpu.VMEM((NP,), jnp.float32),
        pltpu.VMEM((CHUNK,), jnp.int32),
        pltpu.VMEM((CHUNK,), jnp.int32),
        pltpu.VMEM((CHUNK,), jnp.float32),
        pltpu.VMEM((CHUNK,), jnp.float32),
    ],
    **_MESH,
)


def _t1_body(d0_ref, d1_ref, x_ref, bd1_ref, dis_ref, zs1_ref):
    deg = d0_ref[...] + d1_ref[...] + 2.0
    dis = jnp.where(deg > 0, lax.rsqrt(deg), 0.0)
    dis_ref[...] = dis
    z = jnp.dot(x_ref[...], bd1_ref[...], preferred_element_type=jnp.float32)
    zs1_ref[...] = dis * z


_t1_call = pl.pallas_call(
    _t1_body,
    grid=(RW // BRW,),
    in_specs=[
        pl.BlockSpec((BRW, 128), lambda i: (i, 0)),
        pl.BlockSpec((BRW, 128), lambda i: (i, 0)),
        pl.BlockSpec((BRW, 128), lambda i: (i, 0)),
        pl.BlockSpec((128, 128), lambda i: (0, 0)),
    ],
    out_specs=[
        pl.BlockSpec((BRW, 128), lambda i: (i, 0)),
        pl.BlockSpec((BRW, 128), lambda i: (i, 0)),
    ],
    out_shape=[
        jax.ShapeDtypeStruct((RW, 128), jnp.float32),
        jax.ShapeDtypeStruct((RW, 128), jnp.float32),
    ],
)


def _t2_body(a0_ref, a1_ref, zs_ref, dis_ref, b_ref, bd_ref, zsn_ref):
    dis = dis_ref[...]
    pre = dis * (a0_ref[...] + a1_ref[...] + 2.0 * zs_ref[...]) + b_ref[...]
    h = jnp.maximum(pre, 0.0)
    zsn_ref[...] = dis * jnp.dot(h, bd_ref[...],
                                 preferred_element_type=jnp.float32)


_t2_call = pl.pallas_call(
    _t2_body,
    grid=(RW // BRW,),
    in_specs=[
        pl.BlockSpec((BRW, 128), lambda i: (i, 0)),
        pl.BlockSpec((BRW, 128), lambda i: (i, 0)),
        pl.BlockSpec((BRW, 128), lambda i: (i, 0)),
        pl.BlockSpec((BRW, 128), lambda i: (i, 0)),
        pl.BlockSpec((1, 128), lambda i: (0, 0)),
        pl.BlockSpec((128, 128), lambda i: (0, 0)),
    ],
    out_specs=pl.BlockSpec((BRW, 128), lambda i: (i, 0)),
    out_shape=jax.ShapeDtypeStruct((RW, 128), jnp.float32),
)


def _t3_body(a0_ref, a1_ref, zs_ref, dis_ref, b_ref, bd3_ref, zs3_ref, ps_ref):
    i = pl.program_id(0)
    dis = dis_ref[...]
    pre = dis * (a0_ref[...] + a1_ref[...] + 2.0 * zs_ref[...]) + b_ref[...]
    h = jnp.maximum(pre, 0.0)
    u = lax.broadcasted_iota(jnp.int32, (BRW, 128), 0) + i * BRW
    l = lax.broadcasted_iota(jnp.int32, (BRW, 128), 1)
    node = u * 8 + l // 16
    hm = jnp.where(node < NN, h, 0.0)
    zs3_ref[...] = dis * jnp.dot(h, bd3_ref[...],
                                 preferred_element_type=jnp.float32)

    @pl.when(i == 0)
    def _():
        ps_ref[...] = jnp.zeros((1, 128), jnp.float32)

    ps_ref[...] += jnp.sum(hm, axis=0, keepdims=True)


_t3_call = pl.pallas_call(
    _t3_body,
    grid=(RW // BRW,),
    in_specs=[
        pl.BlockSpec((BRW, 128), lambda i: (i, 0)),
        pl.BlockSpec((BRW, 128), lambda i: (i, 0)),
        pl.BlockSpec((BRW, 128), lambda i: (i, 0)),
        pl.BlockSpec((BRW, 128), lambda i: (i, 0)),
        pl.BlockSpec((1, 128), lambda i: (0, 0)),
        pl.BlockSpec((128, 128), lambda i: (0, 0)),
    ],
    out_specs=[
        pl.BlockSpec((BRW, 128), lambda i: (i, 0)),
        pl.BlockSpec((1, 128), lambda i: (0, 0)),
    ],
    out_shape=[
        jax.ShapeDtypeStruct((RW, 128), jnp.float32),
        jax.ShapeDtypeStruct((1, 128), jnp.float32),
    ],
)


def _kc_body(c0_ref, c1_ref, zs3_ref, d0_ref, d1_ref, b3_ref, ch_ref,
             lg_ref, m_ref):
    i = pl.program_id(0)
    deg = d0_ref[...] + d1_ref[...] + 2.0
    dis = jnp.where(deg > 0, lax.rsqrt(deg), 0.0)
    c = dis * (c0_ref[...] + c1_ref[...] + 2.0 * zs3_ref[...]) + b3_ref[...]
    lg = jnp.where(ch_ref[...], c, -1e9)
    lg_ref[...] = lg

    @pl.when(i == 0)
    def _():
        m_ref[...] = jnp.full((1, 1), -3e38, jnp.float32)

    m_ref[...] = jnp.maximum(m_ref[...], jnp.max(lg, keepdims=True))


_kc_call = pl.pallas_call(
    _kc_body,
    grid=(RC // BRC,),
    in_specs=[
        pl.BlockSpec((BRC, 128), lambda i: (i, 0)),
        pl.BlockSpec((BRC, 128), lambda i: (i, 0)),
        pl.BlockSpec((BRC, 128), lambda i: (i, 0)),
        pl.BlockSpec((BRC, 128), lambda i: (i, 0)),
        pl.BlockSpec((BRC, 128), lambda i: (i, 0)),
        pl.BlockSpec((1, 1), lambda i: (0, 0)),
        pl.BlockSpec((BRC, 128), lambda i: (i, 0)),
    ],
    out_specs=[
        pl.BlockSpec((BRC, 128), lambda i: (i, 0)),
        pl.BlockSpec((1, 1), lambda i: (0, 0)),
    ],
    out_shape=[
        jax.ShapeDtypeStruct((RC, 128), jnp.float32),
        jax.ShapeDtypeStruct((1, 1), jnp.float32),
    ],
)


def _ke_body(lg_ref, m_ref, e_ref, s_ref):
    i = pl.program_id(0)
    e = jnp.exp(lg_ref[...] - m_ref[...])
    e_ref[...] = e

    @pl.when(i == 0)
    def _():
        s_ref[...] = jnp.zeros((1, 1), jnp.float32)

    s_ref[...] += jnp.sum(e, keepdims=True)


_ke_call = pl.pallas_call(
    _ke_body,
    grid=(RC // BRC,),
    in_specs=[
        pl.BlockSpec((BRC, 128), lambda i: (i, 0)),
        pl.BlockSpec((1, 1), lambda i: (0, 0)),
    ],
    out_specs=[
        pl.BlockSpec((BRC, 128), lambda i: (i, 0)),
        pl.BlockSpec((1, 1), lambda i: (0, 0)),
    ],
    out_shape=[
        jax.ShapeDtypeStruct((RC, 128), jnp.float32),
        jax.ShapeDtypeStruct((1, 1), jnp.float32),
    ],
)


def _kf_body(e_ref, s_ref, ch_ref, ps_ref, fcw_ref, fcb_ref,
             choice_ref, val_ref):
    i = pl.program_id(0)
    p = e_ref[...] / s_ref[...]
    choice_ref[...] = jnp.where(ch_ref[...], p, 0.0)

    @pl.when(i == 0)
    def _():
        v = jnp.sum(ps_ref[...] * fcw_ref[...], keepdims=True) * (1.0 / NN)
        val_ref[...] = v + fcb_ref[...]


_kf_call = pl.pallas_call(
    _kf_body,
    grid=(RC // BRC,),
    in_specs=[
        pl.BlockSpec((BRC, 128), lambda i: (i, 0)),
        pl.BlockSpec((1, 1), lambda i: (0, 0)),
        pl.BlockSpec((BRC, 128), lambda i: (i, 0)),
        pl.BlockSpec((1, 128), lambda i: (0, 0)),
        pl.BlockSpec((1, 128), lambda i: (0, 0)),
        pl.BlockSpec((1, 1), lambda i: (0, 0)),
    ],
    out_specs=[
        pl.BlockSpec((BRC, 128), lambda i: (i, 0)),
        pl.BlockSpec((1, 1), lambda i: (0, 0)),
    ],
    out_shape=[
        jax.ShapeDtypeStruct((RC, 128), jnp.float32),
        jax.ShapeDtypeStruct((1, 1), jnp.float32),
    ],
)


@jax.jit
def _run(x, edge_attr, W1, b1, W2, b2, W3, b3, fc_W, fc_b, edge_index, choices):
    f32 = jnp.float32
    row = edge_index[0]
    col = edge_index[1]

    xw = jnp.pad(x, ((0, NP - NN), (0, 13))).reshape(RW, 128)
    eye8 = jnp.eye(8, dtype=f32)
    bd1 = jnp.kron(eye8, jnp.pad(W1, ((0, 13), (0, 0))))
    bd2 = jnp.kron(eye8, W2)
    bd3 = jnp.kron(eye8, jnp.pad(W3, ((0, 0), (0, 15))))
    b1w = jnp.tile(b1, 8).reshape(1, 128)
    b2w = jnp.tile(b2, 8).reshape(1, 128)
    fcww = jnp.tile(fc_W[:, 0], 8).reshape(1, 128)
    chc = jnp.pad(choices, (0, NP - NN)).reshape(RC, 128)

    degw, degc = _deg_call(col, edge_attr)
    d0w = degw[:NP].reshape(RW, 128)
    d1w = degw[NP:].reshape(RW, 128)

    dis_w, zs1w = _t1_call(d0w, d1w, xw, bd1)
    acc1 = _pass16_call(row, col, edge_attr, zs1w.reshape(NP, 16))
    zs2w = _t2_call(acc1[:NP].reshape(RW, 128), acc1[NP:].reshape(RW, 128),
                    zs1w, dis_w, b1w, bd2)
    acc2 = _pass16_call(row, col, edge_attr, zs2w.reshape(NP, 16))
    zs3w, ps = _t3_call(acc2[:NP].reshape(RW, 128), acc2[NP:].reshape(RW, 128),
                        zs2w, dis_w, b2w, bd3)

    zs3c = zs3w.reshape(NP, 16)[:, 0]
    cacc = _pass1_call(row, col, edge_attr, zs3c)

    lg, m = _kc_call(cacc[:NP].reshape(RC, 128), cacc[NP:].reshape(RC, 128),
                     zs3c.reshape(RC, 128),
                     degc[:NP].reshape(RC, 128), degc[NP:].reshape(RC, 128),
                     b3.reshape(1, 1), chc)
    e, s = _ke_call(lg, m)
    choice, value = _kf_call(e, s, chc, ps, fcww, fc_b.reshape(1, 1))
    return choice.reshape(NP)[:NN], value


def kernel(x, edge_attr, W1, b1, W2, b2, W3, b3, fc_W, fc_b, edge_index, choices):
    return _run(x, edge_attr, W1, b1, W2, b2, W3, b3, fc_W, fc_b,
                edge_index, choices)


# speedup vs baseline: 58.0075x; 1.2488x over previous
import jax
import jax.numpy as jnp
from jax import lax
from jax.experimental import pallas as pl
from jax.experimental.pallas import tpu as pltpu
from jax.experimental.pallas import tpu_sc as plsc

NN = 100000
EE = 3200000
NC, NS, LANES = 2, 16, 16
NW = NC * NS
NP = 102400
EPT = EE // NW
CHUNK = 800
NCH = EPT // CHUNK
KCH = 400
KNCH = EPT // KCH
NPT = NP // NS
WCH = 800
RW = NP // 8
BRW = 1600
RC = NP // 128
BRC = 200

_SC_PARAMS = pltpu.CompilerParams(use_tc_tiling_on_sc=False,
                                  needs_layout_passes=False)
_MESH = dict(mesh=plsc.VectorSubcoreMesh(core_axis_name="c",
                                         subcore_axis_name="s"),
             compiler_params=_SC_PARAMS)


def _sc_deg_body(col_hbm, ew_hbm, outw_hbm, outc_hbm, acc, idxb, valb, zb, wbuf):
    cid = lax.axis_index("c")
    sid = lax.axis_index("s")
    wid = sid * NC + cid

    def zb_body(j, _):
        zb[pl.ds(j * LANES, LANES)] = jnp.zeros((LANES,), jnp.float32)
        return 0
    lax.fori_loop(0, WCH // LANES, zb_body, 0)

    def zs_body(j, _):
        pltpu.sync_copy(zb, acc.at[pl.ds(sid * NPT + j * WCH, WCH)])
        return 0
    lax.fori_loop(0, NPT // WCH, zs_body, 0)
    plsc.subcore_barrier()

    def ch_body(i, _):
        base = wid * EPT + i * CHUNK
        pltpu.sync_copy(col_hbm.at[pl.ds(base, CHUNK)], idxb)
        pltpu.sync_copy(ew_hbm.at[pl.ds(base, CHUNK)], valb)
        pltpu.sync_copy(valb, acc.at[idxb], add=True)
        return 0
    lax.fori_loop(0, NCH, ch_body, 0)
    plsc.subcore_barrier()

    def wb_body(j, _):
        off = sid * NPT + j * WCH
        pltpu.sync_copy(acc.at[pl.ds(off, WCH)], zb)
        pltpu.sync_copy(zb, outc_hbm.at[pl.ds(cid * NP + off, WCH)])

        def expand(g, _):
            dv = zb[pl.ds(g * LANES, LANES)]
            for k in range(LANES):
                wbuf[g * LANES + k] = jnp.full((LANES,), dv[k], jnp.float32)
            return 0
        lax.fori_loop(0, WCH // LANES, expand, 0)
        pltpu.sync_copy(wbuf, outw_hbm.at[pl.ds(cid * NP + off, WCH)])
        return 0
    lax.fori_loop(0, NPT // WCH, wb_body, 0)


_deg_call = pl.kernel(
    _sc_deg_body,
    out_type=[jax.ShapeDtypeStruct((2 * NP, 16), jnp.float32),
              jax.ShapeDtypeStruct((2 * NP,), jnp.float32)],
    scratch_types=[
        pltpu.VMEM_SHARED((NP,), jnp.float32),
        pltpu.VMEM((CHUNK,), jnp.int32),
        pltpu.VMEM((CHUNK,), jnp.float32),
        pltpu.VMEM((WCH,), jnp.float32),
        pltpu.VMEM((WCH, 16), jnp.float32),
    ],
    **_MESH,
)


def _sc_pass16_body(row_hbm, col_hbm, ew_hbm, zs_hbm, out_hbm,
                    acc, rowb0, colb0, ewb0, rows0,
                    rowb1, colb1, ewb1, rows1,
                    lsem0, lsem1, gsem0, gsem1):
    cid = lax.axis_index("c")
    sid = lax.axis_index("s")
    wid = sid * NC + cid
    ebase = wid * EPT

    def zr(j, _):
        rows0[j] = jnp.zeros((LANES,), jnp.float32)
        return 0
    lax.fori_loop(0, WCH, zr, 0)

    def zs_body(j, _):
        pltpu.sync_copy(rows0.at[pl.ds(0, WCH)],
                        acc.at[pl.ds(sid * NPT + j * WCH, WCH)])
        return 0
    lax.fori_loop(0, NPT // WCH, zs_body, 0)
    plsc.subcore_barrier()

    def issue_lin(j, rowb, colb, ewb, sem):
        b = ebase + j * KCH
        pltpu.async_copy(row_hbm.at[pl.ds(b, KCH)], rowb, sem)
        pltpu.async_copy(col_hbm.at[pl.ds(b, KCH)], colb, sem)
        pltpu.async_copy(ew_hbm.at[pl.ds(b, KCH)], ewb, sem)

    def drain_lin(j, rowb, colb, ewb, sem):
        b = ebase + j * KCH
        pltpu.make_async_copy(row_hbm.at[pl.ds(b, KCH)], rowb, sem).wait()
        pltpu.make_async_copy(col_hbm.at[pl.ds(b, KCH)], colb, sem).wait()
        pltpu.make_async_copy(ew_hbm.at[pl.ds(b, KCH)], ewb, sem).wait()

    def compute(rows, ewb, colb):
        def sc_body(j, _):
            b = j * LANES
            wv = ewb[pl.ds(b, LANES)]
            for k in range(LANES):
                rows[b + k] = rows[b + k] * wv[k]
            return 0
        lax.fori_loop(0, KCH // LANES, sc_body, 0)
        pltpu.sync_copy(rows, acc.at[colb], add=True)

    issue_lin(0, rowb0, colb0, ewb0, lsem0)
    issue_lin(1, rowb1, colb1, ewb1, lsem1)
    drain_lin(0, rowb0, colb0, ewb0, lsem0)
    pltpu.async_copy(zs_hbm.at[rowb0], rows0, gsem0)

    def pair(t, _):
        j0 = 2 * t
        drain_lin(j0 + 1, rowb1, colb1, ewb1, lsem1)
        pltpu.async_copy(zs_hbm.at[rowb1], rows1, gsem1)
        pltpu.make_async_copy(zs_hbm.at[rowb0], rows0, gsem0).wait()
        compute(rows0, ewb0, colb0)

        @pl.when(j0 + 2 < KNCH)
        def _():
            issue_lin(j0 + 2, rowb0, colb0, ewb0, lsem0)

        @pl.when(j0 + 2 < KNCH)
        def _():
            drain_lin(j0 + 2, rowb0, colb0, ewb0, lsem0)
            pltpu.async_copy(zs_hbm.at[rowb0], rows0, gsem0)

        pltpu.make_async_copy(zs_hbm.at[rowb1], rows1, gsem1).wait()
        compute(rows1, ewb1, colb1)

        @pl.when(j0 + 3 < KNCH)
        def _():
            issue_lin(j0 + 3, rowb1, colb1, ewb1, lsem1)
        return 0
    lax.fori_loop(0, KNCH // 2, pair, 0)
    plsc.subcore_barrier()

    def wb(j, _):
        off = sid * NPT + j * WCH
        pltpu.sync_copy(acc.at[pl.ds(off, WCH)], rows0.at[pl.ds(0, WCH)])
        pltpu.sync_copy(rows0.at[pl.ds(0, WCH)],
                        out_hbm.at[pl.ds(cid * NP + off, WCH)])
        return 0
    lax.fori_loop(0, NPT // WCH, wb, 0)


_pass16_call = pl.kernel(
    _sc_pass16_body,
    out_type=jax.ShapeDtypeStruct((2 * NP, 16), jnp.float32),
    scratch_types=[
        pltpu.VMEM_SHARED((NP, 16), jnp.float32),
        pltpu.VMEM((KCH,), jnp.int32),
        pltpu.VMEM((KCH,), jnp.int32),
        pltpu.VMEM((KCH,), jnp.float32),
        pltpu.VMEM((KCH, 16), jnp.float32),
        pltpu.VMEM((KCH,), jnp.int32),
        pltpu.VMEM((KCH,), jnp.int32),
        pltpu.VMEM((KCH,), jnp.float32),
        pltpu.VMEM((KCH, 16), jnp.float32),
        pltpu.SemaphoreType.DMA,
        pltpu.SemaphoreType.DMA,
        pltpu.SemaphoreType.DMA,
        pltpu.SemaphoreType.DMA,
    ],
    **_MESH,
)


def _sc_pass1_body(row_hbm, col_hbm, ew_hbm, zs3_hbm, out_hbm,
                   acc, table, rowb, colb, ewb, scl):
    cid = lax.axis_index("c")
    sid = lax.axis_index("s")
    wid = sid * NC + cid

    pltpu.sync_copy(zs3_hbm, table)

    def zb_body(j, _):
        scl[pl.ds(j * LANES, LANES)] = jnp.zeros((LANES,), jnp.float32)
        return 0
    lax.fori_loop(0, WCH // LANES, zb_body, 0)

    def zs_body(j, _):
        pltpu.sync_copy(scl.at[pl.ds(0, WCH)],
                        acc.at[pl.ds(sid * NPT + j * WCH, WCH)])
        return 0
    lax.fori_loop(0, NPT // WCH, zs_body, 0)
    plsc.subcore_barrier()

    def ch(i, _):
        base = wid * EPT + i * CHUNK
        pltpu.sync_copy(row_hbm.at[pl.ds(base, CHUNK)], rowb)
        pltpu.sync_copy(col_hbm.at[pl.ds(base, CHUNK)], colb)
        pltpu.sync_copy(ew_hbm.at[pl.ds(base, CHUNK)], ewb)

        def g(j, _):
            r = rowb[pl.ds(j * LANES, LANES)]
            v = plsc.load_gather(table, [r])
            w = ewb[pl.ds(j * LANES, LANES)]
            scl[pl.ds(j * LANES, LANES)] = v * w
            return 0
        lax.fori_loop(0, CHUNK // LANES, g, 0)

        pltpu.sync_copy(scl, acc.at[colb], add=True)
        return 0
    lax.fori_loop(0, NCH, ch, 0)
    plsc.subcore_barrier()

    def wb(j, _):
        off = sid * NPT + j * WCH
        pltpu.sync_copy(acc.at[pl.ds(off, WCH)], scl.at[pl.ds(0, WCH)])
        pltpu.sync_copy(scl.at[pl.ds(0, WCH)],
                        out_hbm.at[pl.ds(cid * NP + off, WCH)])
        return 0
    lax.fori_loop(0, NPT // WCH, wb, 0)


_pass1_call = pl.kernel(
    _sc_pass1_body,
    out_type=jax.ShapeDtypeStruct((2 * NP,), jnp.float32),
    scratch_types=[
        pltpu.VMEM_SHARED((NP,), jnp.float32),
        pltpu.VMEM((NP,), jnp.float32),
        pltpu.VMEM((CHUNK,), jnp.int32),
        pltpu.VMEM((CHUNK,), jnp.int32),
        pltpu.VMEM((CHUNK,), jnp.float32),
        pltpu.VMEM((CHUNK,), jnp.float32),
    ],
    **_MESH,
)


def _t1_body(d0_ref, d1_ref, x_ref, bd1_ref, dis_ref, zs1_ref):
    deg = d0_ref[...] + d1_ref[...] + 2.0
    dis = jnp.where(deg > 0, lax.rsqrt(deg), 0.0)
    dis_ref[...] = dis
    z = jnp.dot(x_ref[...], bd1_ref[...], preferred_element_type=jnp.float32)
    zs1_ref[...] = dis * z


_t1_call = pl.pallas_call(
    _t1_body,
    grid=(RW // BRW,),
    in_specs=[
        pl.BlockSpec((BRW, 128), lambda i: (i, 0)),
        pl.BlockSpec((BRW, 128), lambda i: (i, 0)),
        pl.BlockSpec((BRW, 128), lambda i: (i, 0)),
        pl.BlockSpec((128, 128), lambda i: (0, 0)),
    ],
    out_specs=[
        pl.BlockSpec((BRW, 128), lambda i: (i, 0)),
        pl.BlockSpec((BRW, 128), lambda i: (i, 0)),
    ],
    out_shape=[
        jax.ShapeDtypeStruct((RW, 128), jnp.float32),
        jax.ShapeDtypeStruct((RW, 128), jnp.float32),
    ],
)


def _t2_body(a0_ref, a1_ref, zs_ref, dis_ref, b_ref, bd_ref, zsn_ref):
    dis = dis_ref[...]
    pre = dis * (a0_ref[...] + a1_ref[...] + 2.0 * zs_ref[...]) + b_ref[...]
    h = jnp.maximum(pre, 0.0)
    zsn_ref[...] = dis * jnp.dot(h, bd_ref[...],
                                 preferred_element_type=jnp.float32)


_t2_call = pl.pallas_call(
    _t2_body,
    grid=(RW // BRW,),
    in_specs=[
        pl.BlockSpec((BRW, 128), lambda i: (i, 0)),
        pl.BlockSpec((BRW, 128), lambda i: (i, 0)),
        pl.BlockSpec((BRW, 128), lambda i: (i, 0)),
        pl.BlockSpec((BRW, 128), lambda i: (i, 0)),
        pl.BlockSpec((1, 128), lambda i: (0, 0)),
        pl.BlockSpec((128, 128), lambda i: (0, 0)),
    ],
    out_specs=pl.BlockSpec((BRW, 128), lambda i: (i, 0)),
    out_shape=jax.ShapeDtypeStruct((RW, 128), jnp.float32),
)


def _t3_body(a0_ref, a1_ref, zs_ref, dis_ref, b_ref, bd3_ref, zs3_ref, ps_ref):
    i = pl.program_id(0)
    dis = dis_ref[...]
    pre = dis * (a0_ref[...] + a1_ref[...] + 2.0 * zs_ref[...]) + b_ref[...]
    h = jnp.maximum(pre, 0.0)
    u = lax.broadcasted_iota(jnp.int32, (BRW, 128), 0) + i * BRW
    l = lax.broadcasted_iota(jnp.int32, (BRW, 128), 1)
    node = u * 8 + l // 16
    hm = jnp.where(node < NN, h, 0.0)
    zs3_ref[...] = dis * jnp.dot(h, bd3_ref[...],
                                 preferred_element_type=jnp.float32)

    @pl.when(i == 0)
    def _():
        ps_ref[...] = jnp.zeros((1, 128), jnp.float32)

    ps_ref[...] += jnp.sum(hm, axis=0, keepdims=True)


_t3_call = pl.pallas_call(
    _t3_body,
    grid=(RW // BRW,),
    in_specs=[
        pl.BlockSpec((BRW, 128), lambda i: (i, 0)),
        pl.BlockSpec((BRW, 128), lambda i: (i, 0)),
        pl.BlockSpec((BRW, 128), lambda i: (i, 0)),
        pl.BlockSpec((BRW, 128), lambda i: (i, 0)),
        pl.BlockSpec((1, 128), lambda i: (0, 0)),
        pl.BlockSpec((128, 128), lambda i: (0, 0)),
    ],
    out_specs=[
        pl.BlockSpec((BRW, 128), lambda i: (i, 0)),
        pl.BlockSpec((1, 128), lambda i: (0, 0)),
    ],
    out_shape=[
        jax.ShapeDtypeStruct((RW, 128), jnp.float32),
        jax.ShapeDtypeStruct((1, 128), jnp.float32),
    ],
)


def _kc_body(c0_ref, c1_ref, zs3_ref, d0_ref, d1_ref, b3_ref, ch_ref,
             lg_ref, m_ref):
    i = pl.program_id(0)
    deg = d0_ref[...] + d1_ref[...] + 2.0
    dis = jnp.where(deg > 0, lax.rsqrt(deg), 0.0)
    c = dis * (c0_ref[...] + c1_ref[...] + 2.0 * zs3_ref[...]) + b3_ref[...]
    lg = jnp.where(ch_ref[...], c, -1e9)
    lg_ref[...] = lg

    @pl.when(i == 0)
    def _():
        m_ref[...] = jnp.full((1, 1), -3e38, jnp.float32)

    m_ref[...] = jnp.maximum(m_ref[...], jnp.max(lg, keepdims=True))


_kc_call = pl.pallas_call(
    _kc_body,
    grid=(RC // BRC,),
    in_specs=[
        pl.BlockSpec((BRC, 128), lambda i: (i, 0)),
        pl.BlockSpec((BRC, 128), lambda i: (i, 0)),
        pl.BlockSpec((BRC, 128), lambda i: (i, 0)),
        pl.BlockSpec((BRC, 128), lambda i: (i, 0)),
        pl.BlockSpec((BRC, 128), lambda i: (i, 0)),
        pl.BlockSpec((1, 1), lambda i: (0, 0)),
        pl.BlockSpec((BRC, 128), lambda i: (i, 0)),
    ],
    out_specs=[
        pl.BlockSpec((BRC, 128), lambda i: (i, 0)),
        pl.BlockSpec((1, 1), lambda i: (0, 0)),
    ],
    out_shape=[
        jax.ShapeDtypeStruct((RC, 128), jnp.float32),
        jax.ShapeDtypeStruct((1, 1), jnp.float32),
    ],
)


def _ke_body(lg_ref, m_ref, e_ref, s_ref):
    i = pl.program_id(0)
    e = jnp.exp(lg_ref[...] - m_ref[...])
    e_ref[...] = e

    @pl.when(i == 0)
    def _():
        s_ref[...] = jnp.zeros((1, 1), jnp.float32)

    s_ref[...] += jnp.sum(e, keepdims=True)


_ke_call = pl.pallas_call(
    _ke_body,
    grid=(RC // BRC,),
    in_specs=[
        pl.BlockSpec((BRC, 128), lambda i: (i, 0)),
        pl.BlockSpec((1, 1), lambda i: (0, 0)),
    ],
    out_specs=[
        pl.BlockSpec((BRC, 128), lambda i: (i, 0)),
        pl.BlockSpec((1, 1), lambda i: (0, 0)),
    ],
    out_shape=[
        jax.ShapeDtypeStruct((RC, 128), jnp.float32),
        jax.ShapeDtypeStruct((1, 1), jnp.float32),
    ],
)


def _kf_body(e_ref, s_ref, ch_ref, ps_ref, fcw_ref, fcb_ref,
             choice_ref, val_ref):
    i = pl.program_id(0)
    p = e_ref[...] / s_ref[...]
    choice_ref[...] = jnp.where(ch_ref[...], p, 0.0)

    @pl.when(i == 0)
    def _():
        v = jnp.sum(ps_ref[...] * fcw_ref[...], keepdims=True) * (1.0 / NN)
        val_ref[...] = v + fcb_ref[...]


_kf_call = pl.pallas_call(
    _kf_body,
    grid=(RC // BRC,),
    in_specs=[
        pl.BlockSpec((BRC, 128), lambda i: (i, 0)),
        pl.BlockSpec((1, 1), lambda i: (0, 0)),
        pl.BlockSpec((BRC, 128), lambda i: (i, 0)),
        pl.BlockSpec((1, 128), lambda i: (0, 0)),
        pl.BlockSpec((1, 128), lambda i: (0, 0)),
        pl.BlockSpec((1, 1), lambda i: (0, 0)),
    ],
    out_specs=[
        pl.BlockSpec((BRC, 128), lambda i: (i, 0)),
        pl.BlockSpec((1, 1), lambda i: (0, 0)),
    ],
    out_shape=[
        jax.ShapeDtypeStruct((RC, 128), jnp.float32),
        jax.ShapeDtypeStruct((1, 1), jnp.float32),
    ],
)


@jax.jit
def _run(x, edge_attr, W1, b1, W2, b2, W3, b3, fc_W, fc_b, edge_index, choices):
    f32 = jnp.float32
    row = edge_index[0]
    col = edge_index[1]

    xw = jnp.pad(x, ((0, NP - NN), (0, 13))).reshape(RW, 128)
    eye8 = jnp.eye(8, dtype=f32)
    bd1 = jnp.kron(eye8, jnp.pad(W1, ((0, 13), (0, 0))))
    bd2 = jnp.kron(eye8, W2)
    bd3 = jnp.kron(eye8, jnp.pad(W3, ((0, 0), (0, 15))))
    b1w = jnp.tile(b1, 8).reshape(1, 128)
    b2w = jnp.tile(b2, 8).reshape(1, 128)
    fcww = jnp.tile(fc_W[:, 0], 8).reshape(1, 128)
    chc = jnp.pad(choices, (0, NP - NN)).reshape(RC, 128)

    degw, degc = _deg_call(col, edge_attr)
    d0w = degw[:NP].reshape(RW, 128)
    d1w = degw[NP:].reshape(RW, 128)

    dis_w, zs1w = _t1_call(d0w, d1w, xw, bd1)
    acc1 = _pass16_call(row, col, edge_attr, zs1w.reshape(NP, 16))
    zs2w = _t2_call(acc1[:NP].reshape(RW, 128), acc1[NP:].reshape(RW, 128),
                    zs1w, dis_w, b1w, bd2)
    acc2 = _pass16_call(row, col, edge_attr, zs2w.reshape(NP, 16))
    zs3w, ps = _t3_call(acc2[:NP].reshape(RW, 128), acc2[NP:].reshape(RW, 128),
                        zs2w, dis_w, b2w, bd3)

    zs3c = zs3w.reshape(NP, 16)[:, 0]
    cacc = _pass1_call(row, col, edge_attr, zs3c)

    lg, m = _kc_call(cacc[:NP].reshape(RC, 128), cacc[NP:].reshape(RC, 128),
                     zs3c.reshape(RC, 128),
                     degc[:NP].reshape(RC, 128), degc[NP:].reshape(RC, 128),
                     b3.reshape(1, 1), chc)
    e, s = _ke_call(lg, m)
    choice, value = _kf_call(e, s, chc, ps, fcww, fc_b.reshape(1, 1))
    return choice.reshape(NP)[:NN], value


def kernel(x, edge_attr, W1, b1, W2, b2, W3, b3, fc_W, fc_b, edge_index, choices):
    return _run(x, edge_attr, W1, b1, W2, b2, W3, b3, fc_W, fc_b,
                edge_index, choices)


# speedup vs baseline: 63.2757x; 1.0908x over previous
import jax
import jax.numpy as jnp
from jax import lax
from jax.experimental import pallas as pl
from jax.experimental.pallas import tpu as pltpu
from jax.experimental.pallas import tpu_sc as plsc

NN = 100000
EE = 3200000
NC, NS, LANES = 2, 16, 16
NW = NC * NS
NP = 102400
EPT = EE // NW
CHUNK = 800
NCH = EPT // CHUNK
KCH = 400
KNCH = EPT // KCH
NPT = NP // NS
WCH = 800
RW = NP // 8
BRW = 1600
RC = NP // 128
BRC = 200

_SC_PARAMS = pltpu.CompilerParams(use_tc_tiling_on_sc=False,
                                  needs_layout_passes=False)
_MESH = dict(mesh=plsc.VectorSubcoreMesh(core_axis_name="c",
                                         subcore_axis_name="s"),
             compiler_params=_SC_PARAMS)


def _sc_deg_body(col_hbm, ew_hbm, outw_hbm, outc_hbm, acc, idxb, valb, zb, wbuf):
    cid = lax.axis_index("c")
    sid = lax.axis_index("s")
    wid = sid * NC + cid

    def zb_body(j, _):
        zb[pl.ds(j * LANES, LANES)] = jnp.zeros((LANES,), jnp.float32)
        return 0
    lax.fori_loop(0, WCH // LANES, zb_body, 0)

    def zs_body(j, _):
        pltpu.sync_copy(zb, acc.at[pl.ds(sid * NPT + j * WCH, WCH)])
        return 0
    lax.fori_loop(0, NPT // WCH, zs_body, 0)
    plsc.subcore_barrier()

    def ch_body(i, _):
        base = wid * EPT + i * CHUNK
        pltpu.sync_copy(col_hbm.at[pl.ds(base, CHUNK)], idxb)
        pltpu.sync_copy(ew_hbm.at[pl.ds(base, CHUNK)], valb)
        pltpu.sync_copy(valb, acc.at[idxb], add=True)
        return 0
    lax.fori_loop(0, NCH, ch_body, 0)
    plsc.subcore_barrier()

    def wb_body(j, _):
        off = sid * NPT + j * WCH
        pltpu.sync_copy(acc.at[pl.ds(off, WCH)], zb)
        pltpu.sync_copy(zb, outc_hbm.at[pl.ds(cid * NP + off, WCH)])

        def expand(g, _):
            dv = zb[pl.ds(g * LANES, LANES)]
            for k in range(LANES):
                wbuf[g * LANES + k] = jnp.full((LANES,), dv[k], jnp.float32)
            return 0
        lax.fori_loop(0, WCH // LANES, expand, 0)
        pltpu.sync_copy(wbuf, outw_hbm.at[pl.ds(cid * NP + off, WCH)])
        return 0
    lax.fori_loop(0, NPT // WCH, wb_body, 0)


_deg_call = pl.kernel(
    _sc_deg_body,
    out_type=[jax.ShapeDtypeStruct((2 * NP, 16), jnp.float32),
              jax.ShapeDtypeStruct((2 * NP,), jnp.float32)],
    scratch_types=[
        pltpu.VMEM_SHARED((NP,), jnp.float32),
        pltpu.VMEM((CHUNK,), jnp.int32),
        pltpu.VMEM((CHUNK,), jnp.float32),
        pltpu.VMEM((WCH,), jnp.float32),
        pltpu.VMEM((WCH, 16), jnp.float32),
    ],
    **_MESH,
)


def _sc_pass16_body(row_hbm, col_hbm, ew_hbm, zs_hbm, out_hbm,
                    acc, rowb0, colb0, ewb0, rows0,
                    rowb1, colb1, ewb1, rows1,
                    lsem0, lsem1, gsem0, gsem1):
    cid = lax.axis_index("c")
    sid = lax.axis_index("s")
    wid = sid * NC + cid
    ebase = wid * EPT

    def zr(j, _):
        rows0[j] = jnp.zeros((LANES,), jnp.float32)
        return 0
    lax.fori_loop(0, WCH, zr, 0)

    def zs_body(j, _):
        pltpu.sync_copy(rows0.at[pl.ds(0, WCH)],
                        acc.at[pl.ds(sid * NPT + j * WCH, WCH)])
        return 0
    lax.fori_loop(0, NPT // WCH, zs_body, 0)
    plsc.subcore_barrier()

    def issue_lin(j, rowb, colb, ewb, sem):
        b = ebase + j * KCH
        pltpu.async_copy(row_hbm.at[pl.ds(b, KCH)], rowb, sem)
        pltpu.async_copy(col_hbm.at[pl.ds(b, KCH)], colb, sem)
        pltpu.async_copy(ew_hbm.at[pl.ds(b, KCH)], ewb, sem)

    def drain_lin(j, rowb, colb, ewb, sem):
        b = ebase + j * KCH
        pltpu.make_async_copy(row_hbm.at[pl.ds(b, KCH)], rowb, sem).wait()
        pltpu.make_async_copy(col_hbm.at[pl.ds(b, KCH)], colb, sem).wait()
        pltpu.make_async_copy(ew_hbm.at[pl.ds(b, KCH)], ewb, sem).wait()

    def compute(rows, ewb, colb):
        def sc_body(j, _):
            b = j * LANES
            wv = ewb[pl.ds(b, LANES)]
            for k in range(LANES):
                rows[b + k] = rows[b + k] * wv[k]
            return 0
        lax.fori_loop(0, KCH // LANES, sc_body, 0)
        pltpu.sync_copy(rows, acc.at[colb], add=True)

    issue_lin(0, rowb0, colb0, ewb0, lsem0)
    issue_lin(1, rowb1, colb1, ewb1, lsem1)
    drain_lin(0, rowb0, colb0, ewb0, lsem0)
    pltpu.async_copy(zs_hbm.at[rowb0], rows0, gsem0)

    def pair(t, _):
        j0 = 2 * t
        drain_lin(j0 + 1, rowb1, colb1, ewb1, lsem1)
        pltpu.async_copy(zs_hbm.at[rowb1], rows1, gsem1)
        pltpu.make_async_copy(zs_hbm.at[rowb0], rows0, gsem0).wait()
        compute(rows0, ewb0, colb0)

        @pl.when(j0 + 2 < KNCH)
        def _():
            issue_lin(j0 + 2, rowb0, colb0, ewb0, lsem0)

        @pl.when(j0 + 2 < KNCH)
        def _():
            drain_lin(j0 + 2, rowb0, colb0, ewb0, lsem0)
            pltpu.async_copy(zs_hbm.at[rowb0], rows0, gsem0)

        pltpu.make_async_copy(zs_hbm.at[rowb1], rows1, gsem1).wait()
        compute(rows1, ewb1, colb1)

        @pl.when(j0 + 3 < KNCH)
        def _():
            issue_lin(j0 + 3, rowb1, colb1, ewb1, lsem1)
        return 0
    lax.fori_loop(0, KNCH // 2, pair, 0)
    plsc.subcore_barrier()

    def wb(j, _):
        off = sid * NPT + j * WCH
        pltpu.sync_copy(acc.at[pl.ds(off, WCH)], rows0.at[pl.ds(0, WCH)])
        pltpu.sync_copy(rows0.at[pl.ds(0, WCH)],
                        out_hbm.at[pl.ds(cid * NP + off, WCH)])
        return 0
    lax.fori_loop(0, NPT // WCH, wb, 0)


_pass16_call = pl.kernel(
    _sc_pass16_body,
    out_type=jax.ShapeDtypeStruct((2 * NP, 16), jnp.float32),
    scratch_types=[
        pltpu.VMEM_SHARED((NP, 16), jnp.float32),
        pltpu.VMEM((KCH,), jnp.int32),
        pltpu.VMEM((KCH,), jnp.int32),
        pltpu.VMEM((KCH,), jnp.float32),
        pltpu.VMEM((KCH, 16), jnp.float32),
        pltpu.VMEM((KCH,), jnp.int32),
        pltpu.VMEM((KCH,), jnp.int32),
        pltpu.VMEM((KCH,), jnp.float32),
        pltpu.VMEM((KCH, 16), jnp.float32),
        pltpu.SemaphoreType.DMA,
        pltpu.SemaphoreType.DMA,
        pltpu.SemaphoreType.DMA,
        pltpu.SemaphoreType.DMA,
    ],
    **_MESH,
)


def _sc_pass1_body(row_hbm, col_hbm, ew_hbm, zs3_hbm, out_hbm,
                   acc, table, rowb, colb, ewb, scl):
    cid = lax.axis_index("c")
    sid = lax.axis_index("s")
    wid = sid * NC + cid

    pltpu.sync_copy(zs3_hbm, table)

    def zb_body(j, _):
        scl[pl.ds(j * LANES, LANES)] = jnp.zeros((LANES,), jnp.float32)
        return 0
    lax.fori_loop(0, WCH // LANES, zb_body, 0)

    def zs_body(j, _):
        pltpu.sync_copy(scl.at[pl.ds(0, WCH)],
                        acc.at[pl.ds(sid * NPT + j * WCH, WCH)])
        return 0
    lax.fori_loop(0, NPT // WCH, zs_body, 0)
    plsc.subcore_barrier()

    def ch(i, _):
        base = wid * EPT + i * CHUNK
        pltpu.sync_copy(row_hbm.at[pl.ds(base, CHUNK)], rowb)
        pltpu.sync_copy(col_hbm.at[pl.ds(base, CHUNK)], colb)
        pltpu.sync_copy(ew_hbm.at[pl.ds(base, CHUNK)], ewb)

        def g(j, _):
            r = rowb[pl.ds(j * LANES, LANES)]
            v = plsc.load_gather(table, [r])
            w = ewb[pl.ds(j * LANES, LANES)]
            scl[pl.ds(j * LANES, LANES)] = v * w
            return 0
        lax.fori_loop(0, CHUNK // LANES, g, 0)

        pltpu.sync_copy(scl, acc.at[colb], add=True)
        return 0
    lax.fori_loop(0, NCH, ch, 0)
    plsc.subcore_barrier()

    def wb(j, _):
        off = sid * NPT + j * WCH
        pltpu.sync_copy(acc.at[pl.ds(off, WCH)], scl.at[pl.ds(0, WCH)])
        pltpu.sync_copy(scl.at[pl.ds(0, WCH)],
                        out_hbm.at[pl.ds(cid * NP + off, WCH)])
        return 0
    lax.fori_loop(0, NPT // WCH, wb, 0)


_pass1_call = pl.kernel(
    _sc_pass1_body,
    out_type=jax.ShapeDtypeStruct((2 * NP,), jnp.float32),
    scratch_types=[
        pltpu.VMEM_SHARED((NP,), jnp.float32),
        pltpu.VMEM((NP,), jnp.float32),
        pltpu.VMEM((CHUNK,), jnp.int32),
        pltpu.VMEM((CHUNK,), jnp.int32),
        pltpu.VMEM((CHUNK,), jnp.float32),
        pltpu.VMEM((CHUNK,), jnp.float32),
    ],
    **_MESH,
)


NB = 6400
NG16 = NP // NB


def _t1_body(d0_ref, d1_ref, x_ref, w1_ref, dis_ref, zs1_ref):
    deg = d0_ref[...] + d1_ref[...] + 2.0
    dis = jnp.where(deg > 0, lax.rsqrt(deg), 0.0)
    dis_ref[...] = dis
    z = jnp.dot(x_ref[...], w1_ref[...], preferred_element_type=jnp.float32)
    zs1_ref[...] = dis * z


_t1_call = pl.pallas_call(
    _t1_body,
    grid=(NG16,),
    in_specs=[
        pl.BlockSpec((NB, 16), lambda i: (i, 0)),
        pl.BlockSpec((NB, 16), lambda i: (NG16 + i, 0)),
        pl.BlockSpec((NB, 16), lambda i: (i, 0)),
        pl.BlockSpec((16, 16), lambda i: (0, 0)),
    ],
    out_specs=[
        pl.BlockSpec((NB, 16), lambda i: (i, 0)),
        pl.BlockSpec((NB, 16), lambda i: (i, 0)),
    ],
    out_shape=[
        jax.ShapeDtypeStruct((NP, 16), jnp.float32),
        jax.ShapeDtypeStruct((NP, 16), jnp.float32),
    ],
)


def _t2_body(a0_ref, a1_ref, zs_ref, dis_ref, b_ref, w_ref, zsn_ref):
    dis = dis_ref[...]
    pre = dis * (a0_ref[...] + a1_ref[...] + 2.0 * zs_ref[...]) + b_ref[...]
    h = jnp.maximum(pre, 0.0)
    zsn_ref[...] = dis * jnp.dot(h, w_ref[...],
                                 preferred_element_type=jnp.float32)


_t2_call = pl.pallas_call(
    _t2_body,
    grid=(NG16,),
    in_specs=[
        pl.BlockSpec((NB, 16), lambda i: (i, 0)),
        pl.BlockSpec((NB, 16), lambda i: (NG16 + i, 0)),
        pl.BlockSpec((NB, 16), lambda i: (i, 0)),
        pl.BlockSpec((NB, 16), lambda i: (i, 0)),
        pl.BlockSpec((1, 16), lambda i: (0, 0)),
        pl.BlockSpec((16, 16), lambda i: (0, 0)),
    ],
    out_specs=pl.BlockSpec((NB, 16), lambda i: (i, 0)),
    out_shape=jax.ShapeDtypeStruct((NP, 16), jnp.float32),
)


def _t3_body(a0_ref, a1_ref, zs_ref, dis_ref, b_ref, w3_ref, zs3_ref, ps_ref):
    i = pl.program_id(0)
    dis = dis_ref[...]
    pre = dis * (a0_ref[...] + a1_ref[...] + 2.0 * zs_ref[...]) + b_ref[...]
    h = jnp.maximum(pre, 0.0)
    node = lax.broadcasted_iota(jnp.int32, (NB, 16), 0) + i * NB
    hm = jnp.where(node < NN, h, 0.0)
    zs3_ref[...] = dis * jnp.dot(h, w3_ref[...],
                                 preferred_element_type=jnp.float32)

    @pl.when(i == 0)
    def _():
        ps_ref[...] = jnp.zeros((1, 16), jnp.float32)

    ps_ref[...] += jnp.sum(hm, axis=0, keepdims=True)


_t3_call = pl.pallas_call(
    _t3_body,
    grid=(NG16,),
    in_specs=[
        pl.BlockSpec((NB, 16), lambda i: (i, 0)),
        pl.BlockSpec((NB, 16), lambda i: (NG16 + i, 0)),
        pl.BlockSpec((NB, 16), lambda i: (i, 0)),
        pl.BlockSpec((NB, 16), lambda i: (i, 0)),
        pl.BlockSpec((1, 16), lambda i: (0, 0)),
        pl.BlockSpec((16, 16), lambda i: (0, 0)),
    ],
    out_specs=[
        pl.BlockSpec((NB, 16), lambda i: (i, 0)),
        pl.BlockSpec((1, 16), lambda i: (0, 0)),
    ],
    out_shape=[
        jax.ShapeDtypeStruct((NP, 16), jnp.float32),
        jax.ShapeDtypeStruct((1, 16), jnp.float32),
    ],
)


def _kc_body(c0_ref, c1_ref, zs3_ref, d0_ref, d1_ref, b3_ref, ch_ref,
             lg_ref, m_ref):
    i = pl.program_id(0)
    deg = d0_ref[...] + d1_ref[...] + 2.0
    dis = jnp.where(deg > 0, lax.rsqrt(deg), 0.0)
    c = dis * (c0_ref[...] + c1_ref[...] + 2.0 * zs3_ref[...]) + b3_ref[...]
    lg = jnp.where(ch_ref[...], c, -1e9)
    lg_ref[...] = lg

    @pl.when(i == 0)
    def _():
        m_ref[...] = jnp.full((1, 1), -3e38, jnp.float32)

    m_ref[...] = jnp.maximum(m_ref[...], jnp.max(lg, keepdims=True))


_kc_call = pl.pallas_call(
    _kc_body,
    grid=(RC // BRC,),
    in_specs=[
        pl.BlockSpec((BRC, 128), lambda i: (i, 0)),
        pl.BlockSpec((BRC, 128), lambda i: (i, 0)),
        pl.BlockSpec((BRC, 128), lambda i: (i, 0)),
        pl.BlockSpec((BRC, 128), lambda i: (i, 0)),
        pl.BlockSpec((BRC, 128), lambda i: (i, 0)),
        pl.BlockSpec((1, 1), lambda i: (0, 0)),
        pl.BlockSpec((BRC, 128), lambda i: (i, 0)),
    ],
    out_specs=[
        pl.BlockSpec((BRC, 128), lambda i: (i, 0)),
        pl.BlockSpec((1, 1), lambda i: (0, 0)),
    ],
    out_shape=[
        jax.ShapeDtypeStruct((RC, 128), jnp.float32),
        jax.ShapeDtypeStruct((1, 1), jnp.float32),
    ],
)


def _ke_body(lg_ref, m_ref, e_ref, s_ref):
    i = pl.program_id(0)
    e = jnp.exp(lg_ref[...] - m_ref[...])
    e_ref[...] = e

    @pl.when(i == 0)
    def _():
        s_ref[...] = jnp.zeros((1, 1), jnp.float32)

    s_ref[...] += jnp.sum(e, keepdims=True)


_ke_call = pl.pallas_call(
    _ke_body,
    grid=(RC // BRC,),
    in_specs=[
        pl.BlockSpec((BRC, 128), lambda i: (i, 0)),
        pl.BlockSpec((1, 1), lambda i: (0, 0)),
    ],
    out_specs=[
        pl.BlockSpec((BRC, 128), lambda i: (i, 0)),
        pl.BlockSpec((1, 1), lambda i: (0, 0)),
    ],
    out_shape=[
        jax.ShapeDtypeStruct((RC, 128), jnp.float32),
        jax.ShapeDtypeStruct((1, 1), jnp.float32),
    ],
)


def _kf_body(e_ref, s_ref, ch_ref, ps_ref, fcw_ref, fcb_ref,
             choice_ref, val_ref):
    i = pl.program_id(0)
    p = e_ref[...] / s_ref[...]
    choice_ref[...] = jnp.where(ch_ref[...], p, 0.0)

    @pl.when(i == 0)
    def _():
        v = jnp.sum(ps_ref[...] * fcw_ref[...], keepdims=True) * (1.0 / NN)
        val_ref[...] = v + fcb_ref[...]


_kf_call = pl.pallas_call(
    _kf_body,
    grid=(RC // BRC,),
    in_specs=[
        pl.BlockSpec((BRC, 128), lambda i: (i, 0)),
        pl.BlockSpec((1, 1), lambda i: (0, 0)),
        pl.BlockSpec((BRC, 128), lambda i: (i, 0)),
        pl.BlockSpec((1, 16), lambda i: (0, 0)),
        pl.BlockSpec((1, 16), lambda i: (0, 0)),
        pl.BlockSpec((1, 1), lambda i: (0, 0)),
    ],
    out_specs=[
        pl.BlockSpec((BRC, 128), lambda i: (i, 0)),
        pl.BlockSpec((1, 1), lambda i: (0, 0)),
    ],
    out_shape=[
        jax.ShapeDtypeStruct((RC, 128), jnp.float32),
        jax.ShapeDtypeStruct((1, 1), jnp.float32),
    ],
)


@jax.jit
def _run(x, edge_attr, W1, b1, W2, b2, W3, b3, fc_W, fc_b, edge_index, choices):
    row = edge_index[0]
    col = edge_index[1]

    xp = jnp.pad(x, ((0, NP - NN), (0, 13)))
    w1p = jnp.pad(W1, ((0, 13), (0, 0)))
    w3p = jnp.pad(W3, ((0, 0), (0, 15)))
    b1r = b1.reshape(1, 16)
    b2r = b2.reshape(1, 16)
    fcw = fc_W[:, 0].reshape(1, 16)
    chc = jnp.pad(choices, (0, NP - NN)).reshape(RC, 128)

    degw, degc = _deg_call(col, edge_attr)

    dis, zs1 = _t1_call(degw, degw, xp, w1p)
    acc1 = _pass16_call(row, col, edge_attr, zs1)
    zs2 = _t2_call(acc1, acc1, zs1, dis, b1r, W2)
    acc2 = _pass16_call(row, col, edge_attr, zs2)
    zs3, ps = _t3_call(acc2, acc2, zs2, dis, b2r, w3p)

    zs3c = zs3[:, 0]
    cacc = _pass1_call(row, col, edge_attr, zs3c)

    lg, m = _kc_call(cacc[:NP].reshape(RC, 128), cacc[NP:].reshape(RC, 128),
                     zs3c.reshape(RC, 128),
                     degc[:NP].reshape(RC, 128), degc[NP:].reshape(RC, 128),
                     b3.reshape(1, 1), chc)
    e, s = _ke_call(lg, m)
    choice, value = _kf_call(e, s, chc, ps, fcw, fc_b.reshape(1, 1))
    return choice.reshape(NP)[:NN], value


def kernel(x, edge_attr, W1, b1, W2, b2, W3, b3, fc_W, fc_b, edge_index, choices):
    return _run(x, edge_attr, W1, b1, W2, b2, W3, b3, fc_W, fc_b,
                edge_index, choices)


# speedup vs baseline: 82.8935x; 1.3100x over previous
import jax
import jax.numpy as jnp
from jax import lax
from jax.experimental import pallas as pl
from jax.experimental.pallas import tpu as pltpu
from jax.experimental.pallas import tpu_sc as plsc

NN = 100000
EE = 3200000
NC, NS, LANES = 2, 16, 16
NW = NC * NS
NP = 102400
EPT = EE // NW
CHUNK = 800
NCH = EPT // CHUNK
KCH = 400
KNCH = EPT // KCH
NPT = NP // NS
WCH = 800
RW = NP // 8
BRW = 1600
RC = NP // 128
BRC = 200

_SC_PARAMS = pltpu.CompilerParams(use_tc_tiling_on_sc=False,
                                  needs_layout_passes=False)
_MESH = dict(mesh=plsc.VectorSubcoreMesh(core_axis_name="c",
                                         subcore_axis_name="s"),
             compiler_params=_SC_PARAMS)


def _sc_deg_body(col_hbm, ew_hbm, outw_hbm, outc_hbm, acc, idxb, valb, zb, wbuf):
    cid = lax.axis_index("c")
    sid = lax.axis_index("s")
    wid = sid * NC + cid

    def zb_body(j, _):
        zb[pl.ds(j * LANES, LANES)] = jnp.zeros((LANES,), jnp.float32)
        return 0
    lax.fori_loop(0, WCH // LANES, zb_body, 0)

    def zs_body(j, _):
        pltpu.sync_copy(zb, acc.at[pl.ds(sid * NPT + j * WCH, WCH)])
        return 0
    lax.fori_loop(0, NPT // WCH, zs_body, 0)
    plsc.subcore_barrier()

    def ch_body(i, _):
        base = wid * EPT + i * CHUNK
        pltpu.sync_copy(col_hbm.at[pl.ds(base, CHUNK)], idxb)
        pltpu.sync_copy(ew_hbm.at[pl.ds(base, CHUNK)], valb)
        pltpu.sync_copy(valb, acc.at[idxb], add=True)
        return 0
    lax.fori_loop(0, NCH, ch_body, 0)
    plsc.subcore_barrier()

    def wb_body(j, _):
        off = sid * NPT + j * WCH
        pltpu.sync_copy(acc.at[pl.ds(off, WCH)], zb)
        pltpu.sync_copy(zb, outc_hbm.at[pl.ds(cid * NP + off, WCH)])

        def expand(g, _):
            dv = zb[pl.ds(g * LANES, LANES)]
            for k in range(LANES):
                wbuf[g * LANES + k] = jnp.full((LANES,), dv[k], jnp.float32)
            return 0
        lax.fori_loop(0, WCH // LANES, expand, 0)
        pltpu.sync_copy(wbuf, outw_hbm.at[pl.ds(cid * NP + off, WCH)])
        return 0
    lax.fori_loop(0, NPT // WCH, wb_body, 0)


_deg_call = pl.kernel(
    _sc_deg_body,
    out_type=[jax.ShapeDtypeStruct((2 * NP, 16), jnp.float32),
              jax.ShapeDtypeStruct((2 * NP,), jnp.float32)],
    scratch_types=[
        pltpu.VMEM_SHARED((NP,), jnp.float32),
        pltpu.VMEM((CHUNK,), jnp.int32),
        pltpu.VMEM((CHUNK,), jnp.float32),
        pltpu.VMEM((WCH,), jnp.float32),
        pltpu.VMEM((WCH, 16), jnp.float32),
    ],
    **_MESH,
)


def _sc_pass16_body(row_hbm, col_hbm, ew_hbm, zs_hbm, out_hbm,
                    acc, rowb0, colb0, ewb0, rows0,
                    rowb1, colb1, ewb1, rows1,
                    lsem0, lsem1, gsem0, gsem1):
    cid = lax.axis_index("c")
    sid = lax.axis_index("s")
    wid = sid * NC + cid
    ebase = wid * EPT

    def zr(j, _):
        rows0[j] = jnp.zeros((LANES,), jnp.float32)
        return 0
    lax.fori_loop(0, WCH, zr, 0)

    def zs_body(j, _):
        pltpu.sync_copy(rows0.at[pl.ds(0, WCH)],
                        acc.at[pl.ds(sid * NPT + j * WCH, WCH)])
        return 0
    lax.fori_loop(0, NPT // WCH, zs_body, 0)
    plsc.subcore_barrier()

    def issue_lin(j, rowb, colb, ewb, sem):
        b = ebase + j * KCH
        pltpu.async_copy(row_hbm.at[pl.ds(b, KCH)], rowb, sem)
        pltpu.async_copy(col_hbm.at[pl.ds(b, KCH)], colb, sem)
        pltpu.async_copy(ew_hbm.at[pl.ds(b, KCH)], ewb, sem)

    def drain_lin(j, rowb, colb, ewb, sem):
        b = ebase + j * KCH
        pltpu.make_async_copy(row_hbm.at[pl.ds(b, KCH)], rowb, sem).wait()
        pltpu.make_async_copy(col_hbm.at[pl.ds(b, KCH)], colb, sem).wait()
        pltpu.make_async_copy(ew_hbm.at[pl.ds(b, KCH)], ewb, sem).wait()

    def compute(rows, ewb, colb):
        def sc_body(j, _):
            b = j * LANES
            wv = ewb[pl.ds(b, LANES)]
            for k in range(LANES):
                rows[b + k] = rows[b + k] * wv[k]
            return 0
        lax.fori_loop(0, KCH // LANES, sc_body, 0)
        pltpu.sync_copy(rows, acc.at[colb], add=True)

    issue_lin(0, rowb0, colb0, ewb0, lsem0)
    issue_lin(1, rowb1, colb1, ewb1, lsem1)
    drain_lin(0, rowb0, colb0, ewb0, lsem0)
    pltpu.async_copy(zs_hbm.at[rowb0], rows0, gsem0)

    def pair(t, _):
        j0 = 2 * t
        drain_lin(j0 + 1, rowb1, colb1, ewb1, lsem1)
        pltpu.async_copy(zs_hbm.at[rowb1], rows1, gsem1)
        pltpu.make_async_copy(zs_hbm.at[rowb0], rows0, gsem0).wait()
        compute(rows0, ewb0, colb0)

        @pl.when(j0 + 2 < KNCH)
        def _():
            issue_lin(j0 + 2, rowb0, colb0, ewb0, lsem0)

        @pl.when(j0 + 2 < KNCH)
        def _():
            drain_lin(j0 + 2, rowb0, colb0, ewb0, lsem0)
            pltpu.async_copy(zs_hbm.at[rowb0], rows0, gsem0)

        pltpu.make_async_copy(zs_hbm.at[rowb1], rows1, gsem1).wait()
        compute(rows1, ewb1, colb1)

        @pl.when(j0 + 3 < KNCH)
        def _():
            issue_lin(j0 + 3, rowb1, colb1, ewb1, lsem1)
        return 0
    lax.fori_loop(0, KNCH // 2, pair, 0)
    plsc.subcore_barrier()

    def wb(j, _):
        off = sid * NPT + j * WCH
        pltpu.sync_copy(acc.at[pl.ds(off, WCH)], rows0.at[pl.ds(0, WCH)])
        pltpu.sync_copy(rows0.at[pl.ds(0, WCH)],
                        out_hbm.at[pl.ds(cid * NP + off, WCH)])
        return 0
    lax.fori_loop(0, NPT // WCH, wb, 0)


_pass16_call = pl.kernel(
    _sc_pass16_body,
    out_type=jax.ShapeDtypeStruct((2 * NP, 16), jnp.float32),
    scratch_types=[
        pltpu.VMEM_SHARED((NP, 16), jnp.float32),
        pltpu.VMEM((KCH,), jnp.int32),
        pltpu.VMEM((KCH,), jnp.int32),
        pltpu.VMEM((KCH,), jnp.float32),
        pltpu.VMEM((KCH, 16), jnp.float32),
        pltpu.VMEM((KCH,), jnp.int32),
        pltpu.VMEM((KCH,), jnp.int32),
        pltpu.VMEM((KCH,), jnp.float32),
        pltpu.VMEM((KCH, 16), jnp.float32),
        pltpu.SemaphoreType.DMA,
        pltpu.SemaphoreType.DMA,
        pltpu.SemaphoreType.DMA,
        pltpu.SemaphoreType.DMA,
    ],
    **_MESH,
)


def _sc_pass1_body(row_hbm, col_hbm, ew_hbm, zs3_hbm, out_hbm,
                   acc, table, rowb, colb, ewb, scl):
    cid = lax.axis_index("c")
    sid = lax.axis_index("s")
    wid = sid * NC + cid

    pltpu.sync_copy(zs3_hbm, table)

    def zb_body(j, _):
        scl[pl.ds(j * LANES, LANES)] = jnp.zeros((LANES,), jnp.float32)
        return 0
    lax.fori_loop(0, WCH // LANES, zb_body, 0)

    def zs_body(j, _):
        pltpu.sync_copy(scl.at[pl.ds(0, WCH)],
                        acc.at[pl.ds(sid * NPT + j * WCH, WCH)])
        return 0
    lax.fori_loop(0, NPT // WCH, zs_body, 0)
    plsc.subcore_barrier()

    def ch(i, _):
        base = wid * EPT + i * CHUNK
        pltpu.sync_copy(row_hbm.at[pl.ds(base, CHUNK)], rowb)
        pltpu.sync_copy(col_hbm.at[pl.ds(base, CHUNK)], colb)
        pltpu.sync_copy(ew_hbm.at[pl.ds(base, CHUNK)], ewb)

        def g(j, _):
            r = rowb[pl.ds(j * LANES, LANES)]
            v = plsc.load_gather(table, [r])
            w = ewb[pl.ds(j * LANES, LANES)]
            scl[pl.ds(j * LANES, LANES)] = v * w
            return 0
        lax.fori_loop(0, CHUNK // LANES, g, 0)

        pltpu.sync_copy(scl, acc.at[colb], add=True)
        return 0
    lax.fori_loop(0, NCH, ch, 0)
    plsc.subcore_barrier()

    def wb(j, _):
        off = sid * NPT + j * WCH
        pltpu.sync_copy(acc.at[pl.ds(off, WCH)], scl.at[pl.ds(0, WCH)])
        pltpu.sync_copy(scl.at[pl.ds(0, WCH)],
                        out_hbm.at[pl.ds(cid * NP + off, WCH)])
        return 0
    lax.fori_loop(0, NPT // WCH, wb, 0)


_pass1_call = pl.kernel(
    _sc_pass1_body,
    out_type=jax.ShapeDtypeStruct((2 * NP,), jnp.float32),
    scratch_types=[
        pltpu.VMEM_SHARED((NP,), jnp.float32),
        pltpu.VMEM((NP,), jnp.float32),
        pltpu.VMEM((CHUNK,), jnp.int32),
        pltpu.VMEM((CHUNK,), jnp.int32),
        pltpu.VMEM((CHUNK,), jnp.float32),
        pltpu.VMEM((CHUNK,), jnp.float32),
    ],
    **_MESH,
)


BLK = BRW * 128
NGW = RW // BRW


def _t1_body(d0_ref, d1_ref, x_ref, bd1_ref, dis_ref, zs1_ref):
    deg = d0_ref[...] + d1_ref[...] + 2.0
    dis = jnp.where(deg > 0, lax.rsqrt(deg), 0.0).reshape(BRW, 128)
    dis_ref[...] = dis
    z = jnp.dot(x_ref[...], bd1_ref[...], preferred_element_type=jnp.float32)
    zs1_ref[...] = (dis * z).reshape(BLK)


_t1_call = pl.pallas_call(
    _t1_body,
    grid=(NGW,),
    in_specs=[
        pl.BlockSpec((BLK,), lambda i: (i,)),
        pl.BlockSpec((BLK,), lambda i: (NGW + i,)),
        pl.BlockSpec((BRW, 128), lambda i: (i, 0)),
        pl.BlockSpec((128, 128), lambda i: (0, 0)),
    ],
    out_specs=[
        pl.BlockSpec((BRW, 128), lambda i: (i, 0)),
        pl.BlockSpec((BLK,), lambda i: (i,)),
    ],
    out_shape=[
        jax.ShapeDtypeStruct((RW, 128), jnp.float32),
        jax.ShapeDtypeStruct((NP * 16,), jnp.float32),
    ],
)


def _t2_body(a0_ref, a1_ref, zs_ref, dis_ref, b_ref, bd_ref, zsn_ref):
    dis = dis_ref[...]
    s = (a0_ref[...] + a1_ref[...] + 2.0 * zs_ref[...]).reshape(BRW, 128)
    h = jnp.maximum(dis * s + b_ref[...], 0.0)
    zsn_ref[...] = (dis * jnp.dot(h, bd_ref[...],
                                  preferred_element_type=jnp.float32)
                    ).reshape(BLK)


_t2_call = pl.pallas_call(
    _t2_body,
    grid=(NGW,),
    in_specs=[
        pl.BlockSpec((BLK,), lambda i: (i,)),
        pl.BlockSpec((BLK,), lambda i: (NGW + i,)),
        pl.BlockSpec((BLK,), lambda i: (i,)),
        pl.BlockSpec((BRW, 128), lambda i: (i, 0)),
        pl.BlockSpec((1, 128), lambda i: (0, 0)),
        pl.BlockSpec((128, 128), lambda i: (0, 0)),
    ],
    out_specs=pl.BlockSpec((BLK,), lambda i: (i,)),
    out_shape=jax.ShapeDtypeStruct((NP * 16,), jnp.float32),
)


def _t3_body(a0_ref, a1_ref, zs_ref, dis_ref, b_ref, bd3_ref, zs3_ref, ps_ref):
    i = pl.program_id(0)
    dis = dis_ref[...]
    s = (a0_ref[...] + a1_ref[...] + 2.0 * zs_ref[...]).reshape(BRW, 128)
    h = jnp.maximum(dis * s + b_ref[...], 0.0)
    u = lax.broadcasted_iota(jnp.int32, (BRW, 128), 0) + i * BRW
    l = lax.broadcasted_iota(jnp.int32, (BRW, 128), 1)
    node = u * 8 + l // 16
    hm = jnp.where(node < NN, h, 0.0)
    zs3_ref[...] = (dis * jnp.dot(h, bd3_ref[...],
                                  preferred_element_type=jnp.float32)
                    ).reshape(BLK)

    @pl.when(i == 0)
    def _():
        ps_ref[...] = jnp.zeros((1, 128), jnp.float32)

    ps_ref[...] += jnp.sum(hm, axis=0, keepdims=True)


_t3_call = pl.pallas_call(
    _t3_body,
    grid=(NGW,),
    in_specs=[
        pl.BlockSpec((BLK,), lambda i: (i,)),
        pl.BlockSpec((BLK,), lambda i: (NGW + i,)),
        pl.BlockSpec((BLK,), lambda i: (i,)),
        pl.BlockSpec((BRW, 128), lambda i: (i, 0)),
        pl.BlockSpec((1, 128), lambda i: (0, 0)),
        pl.BlockSpec((128, 128), lambda i: (0, 0)),
    ],
    out_specs=[
        pl.BlockSpec((BLK,), lambda i: (i,)),
        pl.BlockSpec((1, 128), lambda i: (0, 0)),
    ],
    out_shape=[
        jax.ShapeDtypeStruct((NP * 16,), jnp.float32),
        jax.ShapeDtypeStruct((1, 128), jnp.float32),
    ],
)


def _kc_body(c0_ref, c1_ref, zs3_ref, d0_ref, d1_ref, b3_ref, ch_ref,
             lg_ref, m_ref):
    i = pl.program_id(0)
    deg = d0_ref[...] + d1_ref[...] + 2.0
    dis = jnp.where(deg > 0, lax.rsqrt(deg), 0.0)
    c = dis * (c0_ref[...] + c1_ref[...] + 2.0 * zs3_ref[...]) + b3_ref[...]
    lg = jnp.where(ch_ref[...], c, -1e9)
    lg_ref[...] = lg

    @pl.when(i == 0)
    def _():
        m_ref[...] = jnp.full((1, 1), -3e38, jnp.float32)

    m_ref[...] = jnp.maximum(m_ref[...], jnp.max(lg, keepdims=True))


_kc_call = pl.pallas_call(
    _kc_body,
    grid=(RC // BRC,),
    in_specs=[
        pl.BlockSpec((BRC, 128), lambda i: (i, 0)),
        pl.BlockSpec((BRC, 128), lambda i: (i, 0)),
        pl.BlockSpec((BRC, 128), lambda i: (i, 0)),
        pl.BlockSpec((BRC, 128), lambda i: (i, 0)),
        pl.BlockSpec((BRC, 128), lambda i: (i, 0)),
        pl.BlockSpec((1, 1), lambda i: (0, 0)),
        pl.BlockSpec((BRC, 128), lambda i: (i, 0)),
    ],
    out_specs=[
        pl.BlockSpec((BRC, 128), lambda i: (i, 0)),
        pl.BlockSpec((1, 1), lambda i: (0, 0)),
    ],
    out_shape=[
        jax.ShapeDtypeStruct((RC, 128), jnp.float32),
        jax.ShapeDtypeStruct((1, 1), jnp.float32),
    ],
)


def _ke_body(lg_ref, m_ref, e_ref, s_ref):
    i = pl.program_id(0)
    e = jnp.exp(lg_ref[...] - m_ref[...])
    e_ref[...] = e

    @pl.when(i == 0)
    def _():
        s_ref[...] = jnp.zeros((1, 1), jnp.float32)

    s_ref[...] += jnp.sum(e, keepdims=True)


_ke_call = pl.pallas_call(
    _ke_body,
    grid=(RC // BRC,),
    in_specs=[
        pl.BlockSpec((BRC, 128), lambda i: (i, 0)),
        pl.BlockSpec((1, 1), lambda i: (0, 0)),
    ],
    out_specs=[
        pl.BlockSpec((BRC, 128), lambda i: (i, 0)),
        pl.BlockSpec((1, 1), lambda i: (0, 0)),
    ],
    out_shape=[
        jax.ShapeDtypeStruct((RC, 128), jnp.float32),
        jax.ShapeDtypeStruct((1, 1), jnp.float32),
    ],
)


def _kf_body(e_ref, s_ref, ch_ref, ps_ref, fcw_ref, fcb_ref,
             choice_ref, val_ref):
    i = pl.program_id(0)
    p = e_ref[...] / s_ref[...]
    choice_ref[...] = jnp.where(ch_ref[...], p, 0.0)

    @pl.when(i == 0)
    def _():
        v = jnp.sum(ps_ref[...] * fcw_ref[...], keepdims=True) * (1.0 / NN)
        val_ref[...] = v + fcb_ref[...]


_kf_call = pl.pallas_call(
    _kf_body,
    grid=(RC // BRC,),
    in_specs=[
        pl.BlockSpec((BRC, 128), lambda i: (i, 0)),
        pl.BlockSpec((1, 1), lambda i: (0, 0)),
        pl.BlockSpec((BRC, 128), lambda i: (i, 0)),
        pl.BlockSpec((1, 128), lambda i: (0, 0)),
        pl.BlockSpec((1, 128), lambda i: (0, 0)),
        pl.BlockSpec((1, 1), lambda i: (0, 0)),
    ],
    out_specs=[
        pl.BlockSpec((BRC, 128), lambda i: (i, 0)),
        pl.BlockSpec((1, 1), lambda i: (0, 0)),
    ],
    out_shape=[
        jax.ShapeDtypeStruct((RC, 128), jnp.float32),
        jax.ShapeDtypeStruct((1, 1), jnp.float32),
    ],
)


@jax.jit
def _run(x, edge_attr, W1, b1, W2, b2, W3, b3, fc_W, fc_b, edge_index, choices):
    row = edge_index[0]
    col = edge_index[1]

    eye8 = jnp.eye(8, dtype=jnp.float32)
    xp = jnp.pad(x, ((0, NP - NN), (0, 13))).reshape(RW, 128)
    bd1 = jnp.kron(eye8, jnp.pad(W1, ((0, 13), (0, 0))))
    bd2 = jnp.kron(eye8, W2)
    bd3 = jnp.kron(eye8, jnp.pad(W3, ((0, 0), (0, 15))))
    b1r = jnp.tile(b1, 8).reshape(1, 128)
    b2r = jnp.tile(b2, 8).reshape(1, 128)
    fcw = jnp.tile(fc_W[:, 0], 8).reshape(1, 128)
    chc = jnp.pad(choices, (0, NP - NN)).reshape(RC, 128)

    degw, degc = _deg_call(col, edge_attr)
    degwf = degw.reshape(2 * NP * 16)

    dis, zs1 = _t1_call(degwf, degwf, xp, bd1)
    acc1 = _pass16_call(row, col, edge_attr, zs1.reshape(NP, 16))
    acc1f = acc1.reshape(2 * NP * 16)
    zs2 = _t2_call(acc1f, acc1f, zs1, dis, b1r, bd2)
    acc2 = _pass16_call(row, col, edge_attr, zs2.reshape(NP, 16))
    acc2f = acc2.reshape(2 * NP * 16)
    zs3, ps = _t3_call(acc2f, acc2f, zs2, dis, b2r, bd3)

    zs3c = zs3.reshape(NP, 16)[:, 0]
    cacc = _pass1_call(row, col, edge_attr, zs3c)

    lg, m = _kc_call(cacc[:NP].reshape(RC, 128), cacc[NP:].reshape(RC, 128),
                     zs3c.reshape(RC, 128),
                     degc[:NP].reshape(RC, 128), degc[NP:].reshape(RC, 128),
                     b3.reshape(1, 1), chc)
    e, s = _ke_call(lg, m)
    choice, value = _kf_call(e, s, chc, ps, fcw, fc_b.reshape(1, 1))
    return choice.reshape(NP)[:NN], value


def kernel(x, edge_attr, W1, b1, W2, b2, W3, b3, fc_W, fc_b, edge_index, choices):
    return _run(x, edge_attr, W1, b1, W2, b2, W3, b3, fc_W, fc_b,
                edge_index, choices)


# speedup vs baseline: 101.1079x; 1.2197x over previous
import jax
import jax.numpy as jnp
from jax import lax
from jax.experimental import pallas as pl
from jax.experimental.pallas import tpu as pltpu
from jax.experimental.pallas import tpu_sc as plsc

NN = 100000
EE = 3200000
NC, NS, LANES = 2, 16, 16
NW = NC * NS
NP = 102400
EPT = EE // NW
DCH = 400
DNCH = EPT // DCH
KCH = 400
KNCH = EPT // KCH
NPT = NP // NS
WCH = 800
RW = NP // 8
BRW = 1600
RC = NP // 128
BRC = 200

_SC_PARAMS = pltpu.CompilerParams(use_tc_tiling_on_sc=False,
                                  needs_layout_passes=False)
_MESH = dict(mesh=plsc.VectorSubcoreMesh(core_axis_name="c",
                                         subcore_axis_name="s"),
             compiler_params=_SC_PARAMS)


def _sc_deg_body(col_hbm, ew_hbm, outw_hbm, outc_hbm, acc,
                 idxb0, valb0, idxb1, valb1, zb, wbuf, sem0, sem1):
    cid = lax.axis_index("c")
    sid = lax.axis_index("s")
    wid = sid * NC + cid
    ebase = wid * EPT

    def zb_body(j, _):
        zb[pl.ds(j * LANES, LANES)] = jnp.zeros((LANES,), jnp.float32)
        return 0
    lax.fori_loop(0, WCH // LANES, zb_body, 0)

    def zs_body(j, _):
        pltpu.sync_copy(zb, acc.at[pl.ds(sid * NPT + j * WCH, WCH)])
        return 0
    lax.fori_loop(0, NPT // WCH, zs_body, 0)
    plsc.subcore_barrier()

    def issue(j, idxb, valb, sem):
        b = ebase + j * DCH
        pltpu.async_copy(col_hbm.at[pl.ds(b, DCH)], idxb, sem)
        pltpu.async_copy(ew_hbm.at[pl.ds(b, DCH)], valb, sem)

    def drain(j, idxb, valb, sem):
        b = ebase + j * DCH
        pltpu.make_async_copy(col_hbm.at[pl.ds(b, DCH)], idxb, sem).wait()
        pltpu.make_async_copy(ew_hbm.at[pl.ds(b, DCH)], valb, sem).wait()

    issue(0, idxb0, valb0, sem0)
    issue(1, idxb1, valb1, sem1)

    def pair(t, _):
        j0 = 2 * t
        drain(j0, idxb0, valb0, sem0)
        pltpu.sync_copy(valb0, acc.at[idxb0], add=True)

        @pl.when(j0 + 2 < DNCH)
        def _():
            issue(j0 + 2, idxb0, valb0, sem0)

        drain(j0 + 1, idxb1, valb1, sem1)
        pltpu.sync_copy(valb1, acc.at[idxb1], add=True)

        @pl.when(j0 + 3 < DNCH)
        def _():
            issue(j0 + 3, idxb1, valb1, sem1)
        return 0
    lax.fori_loop(0, DNCH // 2, pair, 0)
    plsc.subcore_barrier()

    def wb_body(j, _):
        off = sid * NPT + j * WCH
        pltpu.sync_copy(acc.at[pl.ds(off, WCH)], zb)
        pltpu.sync_copy(zb, outc_hbm.at[pl.ds(cid * NP + off, WCH)])

        def expand(g, _):
            dv = zb[pl.ds(g * LANES, LANES)]
            for k in range(LANES):
                wbuf[g * LANES + k] = jnp.full((LANES,), dv[k], jnp.float32)
            return 0
        lax.fori_loop(0, WCH // LANES, expand, 0)
        pltpu.sync_copy(wbuf, outw_hbm.at[pl.ds(cid * NP + off, WCH)])
        return 0
    lax.fori_loop(0, NPT // WCH, wb_body, 0)


_deg_call = pl.kernel(
    _sc_deg_body,
    out_type=[jax.ShapeDtypeStruct((2 * NP, 16), jnp.float32),
              jax.ShapeDtypeStruct((2 * NP,), jnp.float32)],
    scratch_types=[
        pltpu.VMEM_SHARED((NP,), jnp.float32),
        pltpu.VMEM((DCH,), jnp.int32),
        pltpu.VMEM((DCH,), jnp.float32),
        pltpu.VMEM((DCH,), jnp.int32),
        pltpu.VMEM((DCH,), jnp.float32),
        pltpu.VMEM((WCH,), jnp.float32),
        pltpu.VMEM((WCH, 16), jnp.float32),
        pltpu.SemaphoreType.DMA,
        pltpu.SemaphoreType.DMA,
    ],
    **_MESH,
)


def _sc_pass16_body(row_hbm, col_hbm, ew_hbm, zs_hbm, out_hbm,
                    acc, rowb0, colb0, ewb0, rows0,
                    rowb1, colb1, ewb1, rows1,
                    lsem0, lsem1, gsem0, gsem1):
    cid = lax.axis_index("c")
    sid = lax.axis_index("s")
    wid = sid * NC + cid
    ebase = wid * EPT

    def zr(j, _):
        rows0[j] = jnp.zeros((LANES,), jnp.float32)
        return 0
    lax.fori_loop(0, WCH, zr, 0)

    def zs_body(j, _):
        pltpu.sync_copy(rows0.at[pl.ds(0, WCH)],
                        acc.at[pl.ds(sid * NPT + j * WCH, WCH)])
        return 0
    lax.fori_loop(0, NPT // WCH, zs_body, 0)
    plsc.subcore_barrier()

    def issue_lin(j, rowb, colb, ewb, sem):
        b = ebase + j * KCH
        pltpu.async_copy(row_hbm.at[pl.ds(b, KCH)], rowb, sem)
        pltpu.async_copy(col_hbm.at[pl.ds(b, KCH)], colb, sem)
        pltpu.async_copy(ew_hbm.at[pl.ds(b, KCH)], ewb, sem)

    def drain_lin(j, rowb, colb, ewb, sem):
        b = ebase + j * KCH
        pltpu.make_async_copy(row_hbm.at[pl.ds(b, KCH)], rowb, sem).wait()
        pltpu.make_async_copy(col_hbm.at[pl.ds(b, KCH)], colb, sem).wait()
        pltpu.make_async_copy(ew_hbm.at[pl.ds(b, KCH)], ewb, sem).wait()

    def compute(rows, ewb, colb):
        def sc_body(j, _):
            b = j * LANES
            wv = ewb[pl.ds(b, LANES)]
            for k in range(LANES):
                rows[b + k] = rows[b + k] * wv[k]
            return 0
        lax.fori_loop(0, KCH // LANES, sc_body, 0)
        pltpu.sync_copy(rows, acc.at[colb], add=True)

    issue_lin(0, rowb0, colb0, ewb0, lsem0)
    issue_lin(1, rowb1, colb1, ewb1, lsem1)
    drain_lin(0, rowb0, colb0, ewb0, lsem0)
    pltpu.async_copy(zs_hbm.at[rowb0], rows0, gsem0)

    def pair(t, _):
        j0 = 2 * t
        drain_lin(j0 + 1, rowb1, colb1, ewb1, lsem1)
        pltpu.async_copy(zs_hbm.at[rowb1], rows1, gsem1)
        pltpu.make_async_copy(zs_hbm.at[rowb0], rows0, gsem0).wait()
        compute(rows0, ewb0, colb0)

        @pl.when(j0 + 2 < KNCH)
        def _():
            issue_lin(j0 + 2, rowb0, colb0, ewb0, lsem0)

        @pl.when(j0 + 2 < KNCH)
        def _():
            drain_lin(j0 + 2, rowb0, colb0, ewb0, lsem0)
            pltpu.async_copy(zs_hbm.at[rowb0], rows0, gsem0)

        pltpu.make_async_copy(zs_hbm.at[rowb1], rows1, gsem1).wait()
        compute(rows1, ewb1, colb1)

        @pl.when(j0 + 3 < KNCH)
        def _():
            issue_lin(j0 + 3, rowb1, colb1, ewb1, lsem1)
        return 0
    lax.fori_loop(0, KNCH // 2, pair, 0)
    plsc.subcore_barrier()

    def wb(j, _):
        off = sid * NPT + j * WCH
        pltpu.sync_copy(acc.at[pl.ds(off, WCH)], rows0.at[pl.ds(0, WCH)])
        pltpu.sync_copy(rows0.at[pl.ds(0, WCH)],
                        out_hbm.at[pl.ds(cid * NP + off, WCH)])
        return 0
    lax.fori_loop(0, NPT // WCH, wb, 0)


_pass16_call = pl.kernel(
    _sc_pass16_body,
    out_type=jax.ShapeDtypeStruct((2 * NP, 16), jnp.float32),
    scratch_types=[
        pltpu.VMEM_SHARED((NP, 16), jnp.float32),
        pltpu.VMEM((KCH,), jnp.int32),
        pltpu.VMEM((KCH,), jnp.int32),
        pltpu.VMEM((KCH,), jnp.float32),
        pltpu.VMEM((KCH, 16), jnp.float32),
        pltpu.VMEM((KCH,), jnp.int32),
        pltpu.VMEM((KCH,), jnp.int32),
        pltpu.VMEM((KCH,), jnp.float32),
        pltpu.VMEM((KCH, 16), jnp.float32),
        pltpu.SemaphoreType.DMA,
        pltpu.SemaphoreType.DMA,
        pltpu.SemaphoreType.DMA,
        pltpu.SemaphoreType.DMA,
    ],
    **_MESH,
)


def _sc_pass1_body(row_hbm, col_hbm, ew_hbm, zs3_hbm, out_hbm,
                   acc, table, rowb0, colb0, ewb0, scl0,
                   rowb1, colb1, ewb1, scl1, wbb, sem0, sem1):
    cid = lax.axis_index("c")
    sid = lax.axis_index("s")
    wid = sid * NC + cid
    ebase = wid * EPT

    pltpu.sync_copy(zs3_hbm, table)

    def zb_body(j, _):
        wbb[pl.ds(j * LANES, LANES)] = jnp.zeros((LANES,), jnp.float32)
        return 0
    lax.fori_loop(0, WCH // LANES, zb_body, 0)

    def zs_body(j, _):
        pltpu.sync_copy(wbb, acc.at[pl.ds(sid * NPT + j * WCH, WCH)])
        return 0
    lax.fori_loop(0, NPT // WCH, zs_body, 0)
    plsc.subcore_barrier()

    def issue(j, rowb, colb, ewb, sem):
        b = ebase + j * DCH
        pltpu.async_copy(row_hbm.at[pl.ds(b, DCH)], rowb, sem)
        pltpu.async_copy(col_hbm.at[pl.ds(b, DCH)], colb, sem)
        pltpu.async_copy(ew_hbm.at[pl.ds(b, DCH)], ewb, sem)

    def drain(j, rowb, colb, ewb, sem):
        b = ebase + j * DCH
        pltpu.make_async_copy(row_hbm.at[pl.ds(b, DCH)], rowb, sem).wait()
        pltpu.make_async_copy(col_hbm.at[pl.ds(b, DCH)], colb, sem).wait()
        pltpu.make_async_copy(ew_hbm.at[pl.ds(b, DCH)], ewb, sem).wait()

    def compute(rowb, colb, ewb, scl):
        def g(j, _):
            r = rowb[pl.ds(j * LANES, LANES)]
            v = plsc.load_gather(table, [r])
            w = ewb[pl.ds(j * LANES, LANES)]
            scl[pl.ds(j * LANES, LANES)] = v * w
            return 0
        lax.fori_loop(0, DCH // LANES, g, 0)
        pltpu.sync_copy(scl, acc.at[colb], add=True)

    issue(0, rowb0, colb0, ewb0, sem0)
    issue(1, rowb1, colb1, ewb1, sem1)

    def pair(t, _):
        j0 = 2 * t
        drain(j0, rowb0, colb0, ewb0, sem0)
        compute(rowb0, colb0, ewb0, scl0)

        @pl.when(j0 + 2 < DNCH)
        def _():
            issue(j0 + 2, rowb0, colb0, ewb0, sem0)

        drain(j0 + 1, rowb1, colb1, ewb1, sem1)
        compute(rowb1, colb1, ewb1, scl1)

        @pl.when(j0 + 3 < DNCH)
        def _():
            issue(j0 + 3, rowb1, colb1, ewb1, sem1)
        return 0
    lax.fori_loop(0, DNCH // 2, pair, 0)
    plsc.subcore_barrier()

    def wb(j, _):
        off = sid * NPT + j * WCH
        pltpu.sync_copy(acc.at[pl.ds(off, WCH)], wbb)
        pltpu.sync_copy(wbb, out_hbm.at[pl.ds(cid * NP + off, WCH)])
        return 0
    lax.fori_loop(0, NPT // WCH, wb, 0)


_pass1_call = pl.kernel(
    _sc_pass1_body,
    out_type=jax.ShapeDtypeStruct((2 * NP,), jnp.float32),
    scratch_types=[
        pltpu.VMEM_SHARED((NP,), jnp.float32),
        pltpu.VMEM((NP,), jnp.float32),
        pltpu.VMEM((DCH,), jnp.int32),
        pltpu.VMEM((DCH,), jnp.int32),
        pltpu.VMEM((DCH,), jnp.float32),
        pltpu.VMEM((DCH,), jnp.float32),
        pltpu.VMEM((DCH,), jnp.int32),
        pltpu.VMEM((DCH,), jnp.int32),
        pltpu.VMEM((DCH,), jnp.float32),
        pltpu.VMEM((DCH,), jnp.float32),
        pltpu.VMEM((WCH,), jnp.float32),
        pltpu.SemaphoreType.DMA,
        pltpu.SemaphoreType.DMA,
    ],
    **_MESH,
)


BLK = BRW * 128
NGW = RW // BRW


def _t1_body(d0_ref, d1_ref, x_ref, bd1_ref, dis_ref, zs1_ref):
    deg = d0_ref[...] + d1_ref[...] + 2.0
    dis = jnp.where(deg > 0, lax.rsqrt(deg), 0.0).reshape(BRW, 128)
    dis_ref[...] = dis
    z = jnp.dot(x_ref[...], bd1_ref[...], preferred_element_type=jnp.float32)
    zs1_ref[...] = (dis * z).reshape(BLK)


_t1_call = pl.pallas_call(
    _t1_body,
    grid=(NGW,),
    in_specs=[
        pl.BlockSpec((BLK,), lambda i: (i,)),
        pl.BlockSpec((BLK,), lambda i: (NGW + i,)),
        pl.BlockSpec((BRW, 128), lambda i: (i, 0)),
        pl.BlockSpec((128, 128), lambda i: (0, 0)),
    ],
    out_specs=[
        pl.BlockSpec((BRW, 128), lambda i: (i, 0)),
        pl.BlockSpec((BLK,), lambda i: (i,)),
    ],
    out_shape=[
        jax.ShapeDtypeStruct((RW, 128), jnp.float32),
        jax.ShapeDtypeStruct((NP * 16,), jnp.float32),
    ],
)


def _t2_body(a0_ref, a1_ref, zs_ref, dis_ref, b_ref, bd_ref, zsn_ref):
    dis = dis_ref[...]
    s = (a0_ref[...] + a1_ref[...] + 2.0 * zs_ref[...]).reshape(BRW, 128)
    h = jnp.maximum(dis * s + b_ref[...], 0.0)
    zsn_ref[...] = (dis * jnp.dot(h, bd_ref[...],
                                  preferred_element_type=jnp.float32)
                    ).reshape(BLK)


_t2_call = pl.pallas_call(
    _t2_body,
    grid=(NGW,),
    in_specs=[
        pl.BlockSpec((BLK,), lambda i: (i,)),
        pl.BlockSpec((BLK,), lambda i: (NGW + i,)),
        pl.BlockSpec((BLK,), lambda i: (i,)),
        pl.BlockSpec((BRW, 128), lambda i: (i, 0)),
        pl.BlockSpec((1, 128), lambda i: (0, 0)),
        pl.BlockSpec((128, 128), lambda i: (0, 0)),
    ],
    out_specs=pl.BlockSpec((BLK,), lambda i: (i,)),
    out_shape=jax.ShapeDtypeStruct((NP * 16,), jnp.float32),
)


def _t3_body(a0_ref, a1_ref, zs_ref, dis_ref, b_ref, bd3_ref, zs3_ref, ps_ref):
    i = pl.program_id(0)
    dis = dis_ref[...]
    s = (a0_ref[...] + a1_ref[...] + 2.0 * zs_ref[...]).reshape(BRW, 128)
    h = jnp.maximum(dis * s + b_ref[...], 0.0)
    u = lax.broadcasted_iota(jnp.int32, (BRW, 128), 0) + i * BRW
    l = lax.broadcasted_iota(jnp.int32, (BRW, 128), 1)
    node = u * 8 + l // 16
    hm = jnp.where(node < NN, h, 0.0)
    zs3_ref[...] = (dis * jnp.dot(h, bd3_ref[...],
                                  preferred_element_type=jnp.float32)
                    ).reshape(BLK)

    @pl.when(i == 0)
    def _():
        ps_ref[...] = jnp.zeros((1, 128), jnp.float32)

    ps_ref[...] += jnp.sum(hm, axis=0, keepdims=True)


_t3_call = pl.pallas_call(
    _t3_body,
    grid=(NGW,),
    in_specs=[
        pl.BlockSpec((BLK,), lambda i: (i,)),
        pl.BlockSpec((BLK,), lambda i: (NGW + i,)),
        pl.BlockSpec((BLK,), lambda i: (i,)),
        pl.BlockSpec((BRW, 128), lambda i: (i, 0)),
        pl.BlockSpec((1, 128), lambda i: (0, 0)),
        pl.BlockSpec((128, 128), lambda i: (0, 0)),
    ],
    out_specs=[
        pl.BlockSpec((BLK,), lambda i: (i,)),
        pl.BlockSpec((1, 128), lambda i: (0, 0)),
    ],
    out_shape=[
        jax.ShapeDtypeStruct((NP * 16,), jnp.float32),
        jax.ShapeDtypeStruct((1, 128), jnp.float32),
    ],
)


def _kc_body(c0_ref, c1_ref, zs3_ref, d0_ref, d1_ref, b3_ref, ch_ref,
             lg_ref, m_ref):
    i = pl.program_id(0)
    deg = d0_ref[...] + d1_ref[...] + 2.0
    dis = jnp.where(deg > 0, lax.rsqrt(deg), 0.0)
    c = dis * (c0_ref[...] + c1_ref[...] + 2.0 * zs3_ref[...]) + b3_ref[...]
    lg = jnp.where(ch_ref[...], c, -1e9)
    lg_ref[...] = lg

    @pl.when(i == 0)
    def _():
        m_ref[...] = jnp.full((1, 1), -3e38, jnp.float32)

    m_ref[...] = jnp.maximum(m_ref[...], jnp.max(lg, keepdims=True))


_kc_call = pl.pallas_call(
    _kc_body,
    grid=(RC // BRC,),
    in_specs=[
        pl.BlockSpec((BRC, 128), lambda i: (i, 0)),
        pl.BlockSpec((BRC, 128), lambda i: (i, 0)),
        pl.BlockSpec((BRC, 128), lambda i: (i, 0)),
        pl.BlockSpec((BRC, 128), lambda i: (i, 0)),
        pl.BlockSpec((BRC, 128), lambda i: (i, 0)),
        pl.BlockSpec((1, 1), lambda i: (0, 0)),
        pl.BlockSpec((BRC, 128), lambda i: (i, 0)),
    ],
    out_specs=[
        pl.BlockSpec((BRC, 128), lambda i: (i, 0)),
        pl.BlockSpec((1, 1), lambda i: (0, 0)),
    ],
    out_shape=[
        jax.ShapeDtypeStruct((RC, 128), jnp.float32),
        jax.ShapeDtypeStruct((1, 1), jnp.float32),
    ],
)


def _ke_body(lg_ref, m_ref, e_ref, s_ref):
    i = pl.program_id(0)
    e = jnp.exp(lg_ref[...] - m_ref[...])
    e_ref[...] = e

    @pl.when(i == 0)
    def _():
        s_ref[...] = jnp.zeros((1, 1), jnp.float32)

    s_ref[...] += jnp.sum(e, keepdims=True)


_ke_call = pl.pallas_call(
    _ke_body,
    grid=(RC // BRC,),
    in_specs=[
        pl.BlockSpec((BRC, 128), lambda i: (i, 0)),
        pl.BlockSpec((1, 1), lambda i: (0, 0)),
    ],
    out_specs=[
        pl.BlockSpec((BRC, 128), lambda i: (i, 0)),
        pl.BlockSpec((1, 1), lambda i: (0, 0)),
    ],
    out_shape=[
        jax.ShapeDtypeStruct((RC, 128), jnp.float32),
        jax.ShapeDtypeStruct((1, 1), jnp.float32),
    ],
)


def _kf_body(e_ref, s_ref, ch_ref, ps_ref, fcw_ref, fcb_ref,
             choice_ref, val_ref):
    i = pl.program_id(0)
    p = e_ref[...] / s_ref[...]
    choice_ref[...] = jnp.where(ch_ref[...], p, 0.0)

    @pl.when(i == 0)
    def _():
        v = jnp.sum(ps_ref[...] * fcw_ref[...], keepdims=True) * (1.0 / NN)
        val_ref[...] = v + fcb_ref[...]


_kf_call = pl.pallas_call(
    _kf_body,
    grid=(RC // BRC,),
    in_specs=[
        pl.BlockSpec((BRC, 128), lambda i: (i, 0)),
        pl.BlockSpec((1, 1), lambda i: (0, 0)),
        pl.BlockSpec((BRC, 128), lambda i: (i, 0)),
        pl.BlockSpec((1, 128), lambda i: (0, 0)),
        pl.BlockSpec((1, 128), lambda i: (0, 0)),
        pl.BlockSpec((1, 1), lambda i: (0, 0)),
    ],
    out_specs=[
        pl.BlockSpec((BRC, 128), lambda i: (i, 0)),
        pl.BlockSpec((1, 1), lambda i: (0, 0)),
    ],
    out_shape=[
        jax.ShapeDtypeStruct((RC, 128), jnp.float32),
        jax.ShapeDtypeStruct((1, 1), jnp.float32),
    ],
)


@jax.jit
def _run(x, edge_attr, W1, b1, W2, b2, W3, b3, fc_W, fc_b, edge_index, choices):
    row = edge_index[0]
    col = edge_index[1]

    eye8 = jnp.eye(8, dtype=jnp.float32)
    xp = jnp.pad(x, ((0, NP - NN), (0, 13))).reshape(RW, 128)
    bd1 = jnp.kron(eye8, jnp.pad(W1, ((0, 13), (0, 0))))
    bd2 = jnp.kron(eye8, W2)
    bd3 = jnp.kron(eye8, jnp.pad(W3, ((0, 0), (0, 15))))
    b1r = jnp.tile(b1, 8).reshape(1, 128)
    b2r = jnp.tile(b2, 8).reshape(1, 128)
    fcw = jnp.tile(fc_W[:, 0], 8).reshape(1, 128)
    chc = jnp.pad(choices, (0, NP - NN)).reshape(RC, 128)

    degw, degc = _deg_call(col, edge_attr)
    degwf = degw.reshape(2 * NP * 16)

    dis, zs1 = _t1_call(degwf, degwf, xp, bd1)
    acc1 = _pass16_call(row, col, edge_attr, zs1.reshape(NP, 16))
    acc1f = acc1.reshape(2 * NP * 16)
    zs2 = _t2_call(acc1f, acc1f, zs1, dis, b1r, bd2)
    acc2 = _pass16_call(row, col, edge_attr, zs2.reshape(NP, 16))
    acc2f = acc2.reshape(2 * NP * 16)
    zs3, ps = _t3_call(acc2f, acc2f, zs2, dis, b2r, bd3)

    zs3c = zs3.reshape(NP, 16)[:, 0]
    cacc = _pass1_call(row, col, edge_attr, zs3c)

    lg, m = _kc_call(cacc[:NP].reshape(RC, 128), cacc[NP:].reshape(RC, 128),
                     zs3c.reshape(RC, 128),
                     degc[:NP].reshape(RC, 128), degc[NP:].reshape(RC, 128),
                     b3.reshape(1, 1), chc)
    e, s = _ke_call(lg, m)
    choice, value = _kf_call(e, s, chc, ps, fcw, fc_b.reshape(1, 1))
    return choice.reshape(NP)[:NN], value


def kernel(x, edge_attr, W1, b1, W2, b2, W3, b3, fc_W, fc_b, edge_index, choices):
    return _run(x, edge_attr, W1, b1, W2, b2, W3, b3, fc_W, fc_b,
                edge_index, choices)


# speedup vs baseline: 101.3211x; 1.0021x over previous
import jax
import jax.numpy as jnp
from jax import lax
from jax.experimental import pallas as pl
from jax.experimental.pallas import tpu as pltpu
from jax.experimental.pallas import tpu_sc as plsc

NN = 100000
EE = 3200000
NC, NS, LANES = 2, 16, 16
NW = NC * NS
NP = 102400
EPT = EE // NW
DCH = 400
DNCH = EPT // DCH
KCH = 400
KNCH = EPT // KCH
NPT = NP // NS
WCH = 800
RW = NP // 8
BRW = 1600
RC = NP // 128
BRC = 200

_SC_PARAMS = pltpu.CompilerParams(use_tc_tiling_on_sc=False,
                                  needs_layout_passes=False)
_MESH = dict(mesh=plsc.VectorSubcoreMesh(core_axis_name="c",
                                         subcore_axis_name="s"),
             compiler_params=_SC_PARAMS)


def _sc_deg_body(col_hbm, ew_hbm, outw_hbm, outc_hbm, acc,
                 idxb0, valb0, idxb1, valb1, zb, wbuf, sem0, sem1):
    cid = lax.axis_index("c")
    sid = lax.axis_index("s")
    wid = sid * NC + cid
    ebase = wid * EPT

    def zb_body(j, _):
        zb[pl.ds(j * LANES, LANES)] = jnp.zeros((LANES,), jnp.float32)
        return 0
    lax.fori_loop(0, WCH // LANES, zb_body, 0)

    def zs_body(j, _):
        pltpu.sync_copy(zb, acc.at[pl.ds(sid * NPT + j * WCH, WCH)])
        return 0
    lax.fori_loop(0, NPT // WCH, zs_body, 0)
    plsc.subcore_barrier()

    def issue(j, idxb, valb, sem):
        b = ebase + j * DCH
        pltpu.async_copy(col_hbm.at[pl.ds(b, DCH)], idxb, sem)
        pltpu.async_copy(ew_hbm.at[pl.ds(b, DCH)], valb, sem)

    def drain(j, idxb, valb, sem):
        b = ebase + j * DCH
        pltpu.make_async_copy(col_hbm.at[pl.ds(b, DCH)], idxb, sem).wait()
        pltpu.make_async_copy(ew_hbm.at[pl.ds(b, DCH)], valb, sem).wait()

    issue(0, idxb0, valb0, sem0)
    issue(1, idxb1, valb1, sem1)

    def pair(t, _):
        j0 = 2 * t
        drain(j0, idxb0, valb0, sem0)
        pltpu.sync_copy(valb0, acc.at[idxb0], add=True)

        @pl.when(j0 + 2 < DNCH)
        def _():
            issue(j0 + 2, idxb0, valb0, sem0)

        drain(j0 + 1, idxb1, valb1, sem1)
        pltpu.sync_copy(valb1, acc.at[idxb1], add=True)

        @pl.when(j0 + 3 < DNCH)
        def _():
            issue(j0 + 3, idxb1, valb1, sem1)
        return 0
    lax.fori_loop(0, DNCH // 2, pair, 0)
    plsc.subcore_barrier()

    def wb_body(j, _):
        off = sid * NPT + j * WCH
        pltpu.sync_copy(acc.at[pl.ds(off, WCH)], zb)
        pltpu.sync_copy(zb, outc_hbm.at[pl.ds(cid * NP + off, WCH)])

        def expand(g, _):
            dv = zb[pl.ds(g * LANES, LANES)]
            for k in range(LANES):
                wbuf[g * LANES + k] = jnp.full((LANES,), dv[k], jnp.float32)
            return 0
        lax.fori_loop(0, WCH // LANES, expand, 0)
        pltpu.sync_copy(wbuf, outw_hbm.at[pl.ds(cid * NP + off, WCH)])
        return 0
    lax.fori_loop(0, NPT // WCH, wb_body, 0)


_deg_call = pl.kernel(
    _sc_deg_body,
    out_type=[jax.ShapeDtypeStruct((2 * NP, 16), jnp.float32),
              jax.ShapeDtypeStruct((2 * NP,), jnp.float32)],
    scratch_types=[
        pltpu.VMEM_SHARED((NP,), jnp.float32),
        pltpu.VMEM((DCH,), jnp.int32),
        pltpu.VMEM((DCH,), jnp.float32),
        pltpu.VMEM((DCH,), jnp.int32),
        pltpu.VMEM((DCH,), jnp.float32),
        pltpu.VMEM((WCH,), jnp.float32),
        pltpu.VMEM((WCH, 16), jnp.float32),
        pltpu.SemaphoreType.DMA,
        pltpu.SemaphoreType.DMA,
    ],
    **_MESH,
)


def _sc_pass16_body(row_hbm, col_hbm, ew_hbm, zs_hbm, out_hbm,
                    acc, rowb0, colb0, ewb0, rows0,
                    rowb1, colb1, ewb1, rows1,
                    lsem0, lsem1, gsem0, gsem1):
    cid = lax.axis_index("c")
    sid = lax.axis_index("s")
    wid = sid * NC + cid
    ebase = wid * EPT

    def zr(j, _):
        rows0[j] = jnp.zeros((LANES,), jnp.float32)
        return 0
    lax.fori_loop(0, KCH, zr, 0)

    def zs_body(j, _):
        pltpu.sync_copy(rows0.at[pl.ds(0, KCH)],
                        acc.at[pl.ds(sid * NPT + j * KCH, KCH)])
        return 0
    lax.fori_loop(0, NPT // KCH, zs_body, 0)
    plsc.subcore_barrier()

    def issue_lin(j, rowb, colb, ewb, sem):
        b = ebase + j * KCH
        pltpu.async_copy(row_hbm.at[pl.ds(b, KCH)], rowb, sem)
        pltpu.async_copy(col_hbm.at[pl.ds(b, KCH)], colb, sem)
        pltpu.async_copy(ew_hbm.at[pl.ds(b, KCH)], ewb, sem)

    def drain_lin(j, rowb, colb, ewb, sem):
        b = ebase + j * KCH
        pltpu.make_async_copy(row_hbm.at[pl.ds(b, KCH)], rowb, sem).wait()
        pltpu.make_async_copy(col_hbm.at[pl.ds(b, KCH)], colb, sem).wait()
        pltpu.make_async_copy(ew_hbm.at[pl.ds(b, KCH)], ewb, sem).wait()

    def compute(rows, ewb, colb):
        def sc_body(j, _):
            b = j * LANES
            wv = ewb[pl.ds(b, LANES)]
            for k in range(LANES):
                rows[b + k] = rows[b + k] * wv[k]
            return 0
        lax.fori_loop(0, KCH // LANES, sc_body, 0)
        pltpu.sync_copy(rows, acc.at[colb], add=True)

    issue_lin(0, rowb0, colb0, ewb0, lsem0)
    issue_lin(1, rowb1, colb1, ewb1, lsem1)
    drain_lin(0, rowb0, colb0, ewb0, lsem0)
    pltpu.async_copy(zs_hbm.at[rowb0], rows0, gsem0)

    def pair(t, _):
        j0 = 2 * t
        drain_lin(j0 + 1, rowb1, colb1, ewb1, lsem1)
        pltpu.async_copy(zs_hbm.at[rowb1], rows1, gsem1)
        pltpu.make_async_copy(zs_hbm.at[rowb0], rows0, gsem0).wait()
        compute(rows0, ewb0, colb0)

        @pl.when(j0 + 2 < KNCH)
        def _():
            issue_lin(j0 + 2, rowb0, colb0, ewb0, lsem0)

        @pl.when(j0 + 2 < KNCH)
        def _():
            drain_lin(j0 + 2, rowb0, colb0, ewb0, lsem0)
            pltpu.async_copy(zs_hbm.at[rowb0], rows0, gsem0)

        pltpu.make_async_copy(zs_hbm.at[rowb1], rows1, gsem1).wait()
        compute(rows1, ewb1, colb1)

        @pl.when(j0 + 3 < KNCH)
        def _():
            issue_lin(j0 + 3, rowb1, colb1, ewb1, lsem1)
        return 0
    lax.fori_loop(0, KNCH // 2, pair, 0)
    plsc.subcore_barrier()

    def wb(j, _):
        off = sid * NPT + j * KCH
        pltpu.sync_copy(acc.at[pl.ds(off, KCH)], rows0.at[pl.ds(0, KCH)])
        pltpu.sync_copy(rows0.at[pl.ds(0, KCH)],
                        out_hbm.at[pl.ds(cid * NP + off, KCH)])
        return 0
    lax.fori_loop(0, NPT // KCH, wb, 0)


_pass16_call = pl.kernel(
    _sc_pass16_body,
    out_type=jax.ShapeDtypeStruct((2 * NP, 16), jnp.float32),
    scratch_types=[
        pltpu.VMEM_SHARED((NP, 16), jnp.float32),
        pltpu.VMEM((KCH,), jnp.int32),
        pltpu.VMEM((KCH,), jnp.int32),
        pltpu.VMEM((KCH,), jnp.float32),
        pltpu.VMEM((KCH, 16), jnp.float32),
        pltpu.VMEM((KCH,), jnp.int32),
        pltpu.VMEM((KCH,), jnp.int32),
        pltpu.VMEM((KCH,), jnp.float32),
        pltpu.VMEM((KCH, 16), jnp.float32),
        pltpu.SemaphoreType.DMA,
        pltpu.SemaphoreType.DMA,
        pltpu.SemaphoreType.DMA,
        pltpu.SemaphoreType.DMA,
    ],
    **_MESH,
)


def _sc_pass1_body(row_hbm, col_hbm, ew_hbm, zs3_hbm, out_hbm,
                   acc, table, rowb0, colb0, ewb0, scl0,
                   rowb1, colb1, ewb1, scl1, wbb, sem0, sem1):
    cid = lax.axis_index("c")
    sid = lax.axis_index("s")
    wid = sid * NC + cid
    ebase = wid * EPT

    pltpu.sync_copy(zs3_hbm, table)

    def zb_body(j, _):
        wbb[pl.ds(j * LANES, LANES)] = jnp.zeros((LANES,), jnp.float32)
        return 0
    lax.fori_loop(0, WCH // LANES, zb_body, 0)

    def zs_body(j, _):
        pltpu.sync_copy(wbb, acc.at[pl.ds(sid * NPT + j * WCH, WCH)])
        return 0
    lax.fori_loop(0, NPT // WCH, zs_body, 0)
    plsc.subcore_barrier()

    def issue(j, rowb, colb, ewb, sem):
        b = ebase + j * DCH
        pltpu.async_copy(row_hbm.at[pl.ds(b, DCH)], rowb, sem)
        pltpu.async_copy(col_hbm.at[pl.ds(b, DCH)], colb, sem)
        pltpu.async_copy(ew_hbm.at[pl.ds(b, DCH)], ewb, sem)

    def drain(j, rowb, colb, ewb, sem):
        b = ebase + j * DCH
        pltpu.make_async_copy(row_hbm.at[pl.ds(b, DCH)], rowb, sem).wait()
        pltpu.make_async_copy(col_hbm.at[pl.ds(b, DCH)], colb, sem).wait()
        pltpu.make_async_copy(ew_hbm.at[pl.ds(b, DCH)], ewb, sem).wait()

    def compute(rowb, colb, ewb, scl):
        def g(j, _):
            r = rowb[pl.ds(j * LANES, LANES)]
            v = plsc.load_gather(table, [r])
            w = ewb[pl.ds(j * LANES, LANES)]
            scl[pl.ds(j * LANES, LANES)] = v * w
            return 0
        lax.fori_loop(0, DCH // LANES, g, 0)
        pltpu.sync_copy(scl, acc.at[colb], add=True)

    issue(0, rowb0, colb0, ewb0, sem0)
    issue(1, rowb1, colb1, ewb1, sem1)

    def pair(t, _):
        j0 = 2 * t
        drain(j0, rowb0, colb0, ewb0, sem0)
        compute(rowb0, colb0, ewb0, scl0)

        @pl.when(j0 + 2 < DNCH)
        def _():
            issue(j0 + 2, rowb0, colb0, ewb0, sem0)

        drain(j0 + 1, rowb1, colb1, ewb1, sem1)
        compute(rowb1, colb1, ewb1, scl1)

        @pl.when(j0 + 3 < DNCH)
        def _():
            issue(j0 + 3, rowb1, colb1, ewb1, sem1)
        return 0
    lax.fori_loop(0, DNCH // 2, pair, 0)
    plsc.subcore_barrier()

    def wb(j, _):
        off = sid * NPT + j * WCH
        pltpu.sync_copy(acc.at[pl.ds(off, WCH)], wbb)
        pltpu.sync_copy(wbb, out_hbm.at[pl.ds(cid * NP + off, WCH)])
        return 0
    lax.fori_loop(0, NPT // WCH, wb, 0)


_pass1_call = pl.kernel(
    _sc_pass1_body,
    out_type=jax.ShapeDtypeStruct((2 * NP,), jnp.float32),
    scratch_types=[
        pltpu.VMEM_SHARED((NP,), jnp.float32),
        pltpu.VMEM((NP,), jnp.float32),
        pltpu.VMEM((DCH,), jnp.int32),
        pltpu.VMEM((DCH,), jnp.int32),
        pltpu.VMEM((DCH,), jnp.float32),
        pltpu.VMEM((DCH,), jnp.float32),
        pltpu.VMEM((DCH,), jnp.int32),
        pltpu.VMEM((DCH,), jnp.int32),
        pltpu.VMEM((DCH,), jnp.float32),
        pltpu.VMEM((DCH,), jnp.float32),
        pltpu.VMEM((WCH,), jnp.float32),
        pltpu.SemaphoreType.DMA,
        pltpu.SemaphoreType.DMA,
    ],
    **_MESH,
)


BLK = BRW * 128
NGW = RW // BRW


def _t1_body(d0_ref, d1_ref, x_ref, bd1_ref, dis_ref, zs1_ref):
    deg = d0_ref[...] + d1_ref[...] + 2.0
    dis = jnp.where(deg > 0, lax.rsqrt(deg), 0.0).reshape(BRW, 128)
    dis_ref[...] = dis
    z = jnp.dot(x_ref[...], bd1_ref[...], preferred_element_type=jnp.float32)
    zs1_ref[...] = (dis * z).reshape(BLK)


_t1_call = pl.pallas_call(
    _t1_body,
    grid=(NGW,),
    in_specs=[
        pl.BlockSpec((BLK,), lambda i: (i,)),
        pl.BlockSpec((BLK,), lambda i: (NGW + i,)),
        pl.BlockSpec((BRW, 128), lambda i: (i, 0)),
        pl.BlockSpec((128, 128), lambda i: (0, 0)),
    ],
    out_specs=[
        pl.BlockSpec((BRW, 128), lambda i: (i, 0)),
        pl.BlockSpec((BLK,), lambda i: (i,)),
    ],
    out_shape=[
        jax.ShapeDtypeStruct((RW, 128), jnp.float32),
        jax.ShapeDtypeStruct((NP * 16,), jnp.float32),
    ],
)


def _t2_body(a0_ref, a1_ref, zs_ref, dis_ref, b_ref, bd_ref, zsn_ref):
    dis = dis_ref[...]
    s = (a0_ref[...] + a1_ref[...] + 2.0 * zs_ref[...]).reshape(BRW, 128)
    h = jnp.maximum(dis * s + b_ref[...], 0.0)
    zsn_ref[...] = (dis * jnp.dot(h, bd_ref[...],
                                  preferred_element_type=jnp.float32)
                    ).reshape(BLK)


_t2_call = pl.pallas_call(
    _t2_body,
    grid=(NGW,),
    in_specs=[
        pl.BlockSpec((BLK,), lambda i: (i,)),
        pl.BlockSpec((BLK,), lambda i: (NGW + i,)),
        pl.BlockSpec((BLK,), lambda i: (i,)),
        pl.BlockSpec((BRW, 128), lambda i: (i, 0)),
        pl.BlockSpec((1, 128), lambda i: (0, 0)),
        pl.BlockSpec((128, 128), lambda i: (0, 0)),
    ],
    out_specs=pl.BlockSpec((BLK,), lambda i: (i,)),
    out_shape=jax.ShapeDtypeStruct((NP * 16,), jnp.float32),
)


def _t3_body(a0_ref, a1_ref, zs_ref, dis_ref, b_ref, bd3_ref, zs3_ref, ps_ref):
    i = pl.program_id(0)
    dis = dis_ref[...]
    s = (a0_ref[...] + a1_ref[...] + 2.0 * zs_ref[...]).reshape(BRW, 128)
    h = jnp.maximum(dis * s + b_ref[...], 0.0)
    u = lax.broadcasted_iota(jnp.int32, (BRW, 128), 0) + i * BRW
    l = lax.broadcasted_iota(jnp.int32, (BRW, 128), 1)
    node = u * 8 + l // 16
    hm = jnp.where(node < NN, h, 0.0)
    zs3_ref[...] = (dis * jnp.dot(h, bd3_ref[...],
                                  preferred_element_type=jnp.float32)
                    ).reshape(BLK)

    @pl.when(i == 0)
    def _():
        ps_ref[...] = jnp.zeros((1, 128), jnp.float32)

    ps_ref[...] += jnp.sum(hm, axis=0, keepdims=True)


_t3_call = pl.pallas_call(
    _t3_body,
    grid=(NGW,),
    in_specs=[
        pl.BlockSpec((BLK,), lambda i: (i,)),
        pl.BlockSpec((BLK,), lambda i: (NGW + i,)),
        pl.BlockSpec((BLK,), lambda i: (i,)),
        pl.BlockSpec((BRW, 128), lambda i: (i, 0)),
        pl.BlockSpec((1, 128), lambda i: (0, 0)),
        pl.BlockSpec((128, 128), lambda i: (0, 0)),
    ],
    out_specs=[
        pl.BlockSpec((BLK,), lambda i: (i,)),
        pl.BlockSpec((1, 128), lambda i: (0, 0)),
    ],
    out_shape=[
        jax.ShapeDtypeStruct((NP * 16,), jnp.float32),
        jax.ShapeDtypeStruct((1, 128), jnp.float32),
    ],
)


def _kc_body(c0_ref, c1_ref, zs3_ref, d0_ref, d1_ref, b3_ref, ch_ref,
             lg_ref, m_ref):
    i = pl.program_id(0)
    deg = d0_ref[...] + d1_ref[...] + 2.0
    dis = jnp.where(deg > 0, lax.rsqrt(deg), 0.0)
    c = dis * (c0_ref[...] + c1_ref[...] + 2.0 * zs3_ref[...]) + b3_ref[...]
    lg = jnp.where(ch_ref[...], c, -1e9)
    lg_ref[...] = lg

    @pl.when(i == 0)
    def _():
        m_ref[...] = jnp.full((1, 1), -3e38, jnp.float32)

    m_ref[...] = jnp.maximum(m_ref[...], jnp.max(lg, keepdims=True))


_kc_call = pl.pallas_call(
    _kc_body,
    grid=(RC // BRC,),
    in_specs=[
        pl.BlockSpec((BRC, 128), lambda i: (i, 0)),
        pl.BlockSpec((BRC, 128), lambda i: (i, 0)),
        pl.BlockSpec((BRC, 128), lambda i: (i, 0)),
        pl.BlockSpec((BRC, 128), lambda i: (i, 0)),
        pl.BlockSpec((BRC, 128), lambda i: (i, 0)),
        pl.BlockSpec((1, 1), lambda i: (0, 0)),
        pl.BlockSpec((BRC, 128), lambda i: (i, 0)),
    ],
    out_specs=[
        pl.BlockSpec((BRC, 128), lambda i: (i, 0)),
        pl.BlockSpec((1, 1), lambda i: (0, 0)),
    ],
    out_shape=[
        jax.ShapeDtypeStruct((RC, 128), jnp.float32),
        jax.ShapeDtypeStruct((1, 1), jnp.float32),
    ],
)


def _ke_body(lg_ref, m_ref, e_ref, s_ref):
    i = pl.program_id(0)
    e = jnp.exp(lg_ref[...] - m_ref[...])
    e_ref[...] = e

    @pl.when(i == 0)
    def _():
        s_ref[...] = jnp.zeros((1, 1), jnp.float32)

    s_ref[...] += jnp.sum(e, keepdims=True)


_ke_call = pl.pallas_call(
    _ke_body,
    grid=(RC // BRC,),
    in_specs=[
        pl.BlockSpec((BRC, 128), lambda i: (i, 0)),
        pl.BlockSpec((1, 1), lambda i: (0, 0)),
    ],
    out_specs=[
        pl.BlockSpec((BRC, 128), lambda i: (i, 0)),
        pl.BlockSpec((1, 1), lambda i: (0, 0)),
    ],
    out_shape=[
        jax.ShapeDtypeStruct((RC, 128), jnp.float32),
        jax.ShapeDtypeStruct((1, 1), jnp.float32),
    ],
)


def _kf_body(e_ref, s_ref, ch_ref, ps_ref, fcw_ref, fcb_ref,
             choice_ref, val_ref):
    i = pl.program_id(0)
    p = e_ref[...] / s_ref[...]
    choice_ref[...] = jnp.where(ch_ref[...], p, 0.0)

    @pl.when(i == 0)
    def _():
        v = jnp.sum(ps_ref[...] * fcw_ref[...], keepdims=True) * (1.0 / NN)
        val_ref[...] = v + fcb_ref[...]


_kf_call = pl.pallas_call(
    _kf_body,
    grid=(RC // BRC,),
    in_specs=[
        pl.BlockSpec((BRC, 128), lambda i: (i, 0)),
        pl.BlockSpec((1, 1), lambda i: (0, 0)),
        pl.BlockSpec((BRC, 128), lambda i: (i, 0)),
        pl.BlockSpec((1, 128), lambda i: (0, 0)),
        pl.BlockSpec((1, 128), lambda i: (0, 0)),
        pl.BlockSpec((1, 1), lambda i: (0, 0)),
    ],
    out_specs=[
        pl.BlockSpec((BRC, 128), lambda i: (i, 0)),
        pl.BlockSpec((1, 1), lambda i: (0, 0)),
    ],
    out_shape=[
        jax.ShapeDtypeStruct((RC, 128), jnp.float32),
        jax.ShapeDtypeStruct((1, 1), jnp.float32),
    ],
)


@jax.jit
def _run(x, edge_attr, W1, b1, W2, b2, W3, b3, fc_W, fc_b, edge_index, choices):
    row = edge_index[0]
    col = edge_index[1]

    eye8 = jnp.eye(8, dtype=jnp.float32)
    xp = jnp.pad(x, ((0, NP - NN), (0, 13))).reshape(RW, 128)
    bd1 = jnp.kron(eye8, jnp.pad(W1, ((0, 13), (0, 0))))
    bd2 = jnp.kron(eye8, W2)
    bd3 = jnp.kron(eye8, jnp.pad(W3, ((0, 0), (0, 15))))
    b1r = jnp.tile(b1, 8).reshape(1, 128)
    b2r = jnp.tile(b2, 8).reshape(1, 128)
    fcw = jnp.tile(fc_W[:, 0], 8).reshape(1, 128)
    chc = jnp.pad(choices, (0, NP - NN)).reshape(RC, 128)

    degw, degc = _deg_call(col, edge_attr)
    degwf = degw.reshape(2 * NP * 16)

    dis, zs1 = _t1_call(degwf, degwf, xp, bd1)
    acc1 = _pass16_call(row, col, edge_attr, zs1.reshape(NP, 16))
    acc1f = acc1.reshape(2 * NP * 16)
    zs2 = _t2_call(acc1f, acc1f, zs1, dis, b1r, bd2)
    acc2 = _pass16_call(row, col, edge_attr, zs2.reshape(NP, 16))
    acc2f = acc2.reshape(2 * NP * 16)
    zs3, ps = _t3_call(acc2f, acc2f, zs2, dis, b2r, bd3)

    zs3c = zs3.reshape(NP, 16)[:, 0]
    cacc = _pass1_call(row, col, edge_attr, zs3c)

    lg, m = _kc_call(cacc[:NP].reshape(RC, 128), cacc[NP:].reshape(RC, 128),
                     zs3c.reshape(RC, 128),
                     degc[:NP].reshape(RC, 128), degc[NP:].reshape(RC, 128),
                     b3.reshape(1, 1), chc)
    e, s = _ke_call(lg, m)
    choice, value = _kf_call(e, s, chc, ps, fcw, fc_b.reshape(1, 1))
    return choice.reshape(NP)[:NN], value


def kernel(x, edge_attr, W1, b1, W2, b2, W3, b3, fc_W, fc_b, edge_index, choices):
    return _run(x, edge_attr, W1, b1, W2, b2, W3, b3, fc_W, fc_b,
                edge_index, choices)
